# Initial kernel scaffold; baseline (speedup 1.0000x reference)
import jax
import jax.numpy as jnp
from jax.experimental import pallas as pl


def kernel(h, m_st, rbf, cbf, idx_s, idx_t, idx_swap, id3_kt, id3_st, id3_ragged_idx, W_skip, W_mkt, W_rbf3, W_down, W_bil, W_up_st, W_up_ts, W_res_before, W_res_after, W_ae_rbf, W_ae_dense, W_ae_res, W_self, W_res_m):
    raise NotImplementedError("write your pallas kernel here")


# speedup vs baseline: 1.3458x; 1.3458x over previous
import functools

import jax
import jax.numpy as jnp
from jax import lax
from jax.experimental import pallas as pl
from jax.experimental.pallas import tpu as pltpu

INV_SQRT_2 = 0.7071067811865475

EB = 1280
TB = 1280
NB = 2000


def _silu(x):
    return x * (1.0 / (1.0 + jnp.exp(-x)))


def _row_spec(rb, cols):
    return pl.BlockSpec((rb, cols), lambda i: (i, 0))


def _full_spec(shape):
    nd = len(shape)
    return pl.BlockSpec(shape, lambda i: (0,) * nd)


def _mkt_body(m_ref, rbf_ref, wmkt_ref, wrbf3_ref, wdown_ref, out_ref):
    t = _silu(jnp.dot(m_ref[...], wmkt_ref[...], preferred_element_type=jnp.float32))
    t = t * jnp.dot(rbf_ref[...], wrbf3_ref[...], preferred_element_type=jnp.float32)
    out_ref[...] = _silu(jnp.dot(t, wdown_ref[...], preferred_element_type=jnp.float32))


def _stage_mkt(m_st, rbf, W_mkt, W_rbf3, W_down):
    E, D = m_st.shape
    K = W_down.shape[1]
    return pl.pallas_call(
        _mkt_body,
        grid=(E // EB,),
        in_specs=[_row_spec(EB, D), _row_spec(EB, rbf.shape[1]),
                  _full_spec(W_mkt.shape), _full_spec(W_rbf3.shape), _full_spec(W_down.shape)],
        out_specs=_row_spec(EB, K),
        out_shape=jax.ShapeDtypeStruct((E, K), jnp.float32),
    )(m_st, rbf, W_mkt, W_rbf3, W_down)


def _bilinear_body(mg_ref, cbf_ref, wb_ref, out_ref):
    mg = mg_ref[...]
    cbf = cbf_ref[...]
    C = cbf.shape[1]
    z = jnp.concatenate([mg * cbf[:, c:c + 1] for c in range(C)], axis=1)
    out_ref[...] = jnp.dot(z, wb_ref[...], preferred_element_type=jnp.float32)


def _stage_bilinear(m_kt_g, cbf, Wb2):
    T, K = m_kt_g.shape
    B = Wb2.shape[1]
    return pl.pallas_call(
        _bilinear_body,
        grid=(T // TB,),
        in_specs=[_row_spec(TB, K), _row_spec(TB, cbf.shape[1]), _full_spec(Wb2.shape)],
        out_specs=_row_spec(TB, B),
        out_shape=jax.ShapeDtypeStruct((T, B), jnp.float32),
    )(m_kt_g, cbf, Wb2)


def _xts_body(x_ref, w_ref, out_ref):
    out_ref[...] = _silu(jnp.dot(x_ref[...], w_ref[...], preferred_element_type=jnp.float32))


def _stage_xts(x, W_up_ts):
    E, B = x.shape
    D = W_up_ts.shape[1]
    return pl.pallas_call(
        _xts_body,
        grid=(E // EB,),
        in_specs=[_row_spec(EB, B), _full_spec(W_up_ts.shape)],
        out_specs=_row_spec(EB, D),
        out_shape=jax.ShapeDtypeStruct((E, D), jnp.float32),
    )(x, W_up_ts)


def _edge_chain_body(m_ref, rbf_ref, x_ref, xts_ref,
                     wskip_ref, wupst_ref, wrb0_ref, wrb1_ref, wra0_ref, wra1_ref,
                     waerbf_ref, m_out_ref, xa_pre_ref):
    m_st = m_ref[...]
    x_skip = jnp.dot(m_st, wskip_ref[...], preferred_element_type=jnp.float32)
    x_st = _silu(jnp.dot(x_ref[...], wupst_ref[...], preferred_element_type=jnp.float32))
    x3 = (x_st + xts_ref[...]) * INV_SQRT_2
    x = (x_skip + x3) * INV_SQRT_2
    y = _silu(jnp.dot(x, wrb0_ref[...], preferred_element_type=jnp.float32))
    y = _silu(jnp.dot(y, wrb1_ref[...], preferred_element_type=jnp.float32))
    x = (x + y) * INV_SQRT_2
    m = (m_st + x) * INV_SQRT_2
    y = _silu(jnp.dot(m, wra0_ref[...], preferred_element_type=jnp.float32))
    y = _silu(jnp.dot(y, wra1_ref[...], preferred_element_type=jnp.float32))
    m = (m + y) * INV_SQRT_2
    m_out_ref[...] = m
    xa_pre_ref[...] = m * jnp.dot(rbf_ref[...], waerbf_ref[...], preferred_element_type=jnp.float32)


def _stage_edge_chain(m_st, rbf, x, xts_sw, W_skip, W_up_st, Wrb0, Wrb1, Wra0, Wra1, W_ae_rbf):
    E, D = m_st.shape
    return pl.pallas_call(
        _edge_chain_body,
        grid=(E // EB,),
        in_specs=[_row_spec(EB, D), _row_spec(EB, rbf.shape[1]), _row_spec(EB, x.shape[1]),
                  _row_spec(EB, D)] + [_full_spec(w.shape) for w in
                                       (W_skip, W_up_st, Wrb0, Wrb1, Wra0, Wra1, W_ae_rbf)],
        out_specs=[_row_spec(EB, D), _row_spec(EB, D)],
        out_shape=[jax.ShapeDtypeStruct((E, D), jnp.float32),
                   jax.ShapeDtypeStruct((E, D), jnp.float32)],
    )(m_st, rbf, x, xts_sw, W_skip, W_up_st, Wrb0, Wrb1, Wra0, Wra1, W_ae_rbf)


def _node_body(xa_ref, h_ref, wd_ref, wa0_ref, wa1_ref, out_ref):
    xa = jnp.sum(xa_ref[...], axis=0)
    xa = _silu(jnp.dot(xa, wd_ref[...], preferred_element_type=jnp.float32))
    y = _silu(jnp.dot(xa, wa0_ref[...], preferred_element_type=jnp.float32))
    y = _silu(jnp.dot(y, wa1_ref[...], preferred_element_type=jnp.float32))
    xa = (xa + y) * INV_SQRT_2
    out_ref[...] = (h_ref[...] + xa) * INV_SQRT_2


def _stage_node(xa_parts, h, W_ae_dense, Wa0, Wa1):
    P, N, D = xa_parts.shape
    A = h.shape[1]
    return pl.pallas_call(
        _node_body,
        grid=(N // NB,),
        in_specs=[pl.BlockSpec((P, NB, D), lambda i: (0, i, 0)), _row_spec(NB, A),
                  _full_spec(W_ae_dense.shape), _full_spec(Wa0.shape), _full_spec(Wa1.shape)],
        out_specs=_row_spec(NB, A),
        out_shape=jax.ShapeDtypeStruct((N, A), jnp.float32),
    )(xa_parts, h, W_ae_dense, Wa0, Wa1)


def _final_body(hs_ref, ht_ref, m_ref, ws0_ref, ws1_ref, ws2_ref, wm0_ref, wm1_ref, out_ref):
    m = m_ref[...]
    m2 = _silu(jnp.dot(hs_ref[...], ws0_ref[...], preferred_element_type=jnp.float32)
               + jnp.dot(ht_ref[...], ws1_ref[...], preferred_element_type=jnp.float32)
               + jnp.dot(m, ws2_ref[...], preferred_element_type=jnp.float32))
    y = _silu(jnp.dot(m2, wm0_ref[...], preferred_element_type=jnp.float32))
    y = _silu(jnp.dot(y, wm1_ref[...], preferred_element_type=jnp.float32))
    m2 = (m2 + y) * INV_SQRT_2
    out_ref[...] = (m + m2) * INV_SQRT_2


def _stage_final(hs, ht, m, Ws0, Ws1, Ws2, Wm0, Wm1):
    E, D = m.shape
    return pl.pallas_call(
        _final_body,
        grid=(E // EB,),
        in_specs=[_row_spec(EB, hs.shape[1]), _row_spec(EB, ht.shape[1]), _row_spec(EB, D)]
                 + [_full_spec(w.shape) for w in (Ws0, Ws1, Ws2, Wm0, Wm1)],
        out_specs=_row_spec(EB, D),
        out_shape=jax.ShapeDtypeStruct((E, D), jnp.float32),
    )(hs, ht, m, Ws0, Ws1, Ws2, Wm0, Wm1)


def kernel(h, m_st, rbf, cbf, idx_s, idx_t, idx_swap, id3_kt, id3_st, id3_ragged_idx,
           W_skip, W_mkt, W_rbf3, W_down, W_bil, W_up_st, W_up_ts,
           W_res_before, W_res_after, W_ae_rbf, W_ae_dense, W_ae_res, W_self, W_res_m):
    E = m_st.shape[0]
    N, A = h.shape
    D = m_st.shape[1]
    C, K, B = W_bil.shape

    Wb2 = W_bil.reshape(C * K, B)
    Wrb0, Wrb1 = W_res_before[0, 0], W_res_before[0, 1]
    Wra0, Wra1 = W_res_after[0, 0], W_res_after[0, 1]
    Wa0, Wa1 = W_ae_res[0, 0], W_ae_res[0, 1]
    Ws0, Ws1, Ws2 = W_self[:A], W_self[A:2 * A], W_self[2 * A:]
    Wm0, Wm1 = W_res_m[0, 0], W_res_m[0, 1]

    m_kt = _stage_mkt(m_st, rbf, W_mkt, W_rbf3, W_down)
    m_kt_g = jnp.take(m_kt, id3_kt, axis=0)
    v = _stage_bilinear(m_kt_g, cbf, Wb2)
    x = jax.ops.segment_sum(v, id3_st, num_segments=E)
    x_ts = _stage_xts(x, W_up_ts)
    xts_sw = jnp.take(x_ts, idx_swap, axis=0)
    m_mid, xa_pre = _stage_edge_chain(m_st, rbf, x, xts_sw,
                                      W_skip, W_up_st, Wrb0, Wrb1, Wra0, Wra1, W_ae_rbf)
    xa = jax.ops.segment_sum(xa_pre, idx_t, num_segments=N)
    h_new = _stage_node(xa[None], h, W_ae_dense, Wa0, Wa1)
    hs = jnp.take(h_new, idx_s, axis=0)
    ht = jnp.take(h_new, idx_t, axis=0)
    m_new = _stage_final(hs, ht, m_mid, Ws0, Ws1, Ws2, Wm0, Wm1)
    return h_new, m_new


# speedup vs baseline: 2.4557x; 1.8247x over previous
import functools

import jax
import jax.numpy as jnp
from jax import lax
from jax.experimental import pallas as pl
from jax.experimental.pallas import tpu as pltpu
from jax.experimental.pallas import tpu_sc as plsc

INV_SQRT_2 = 0.7071067811865475

_SC_MESH = plsc.VectorSubcoreMesh(core_axis_name="c", subcore_axis_name="s")
_NW = 32
_GCH = 128


def _sc_gather(table, idx):
    M = idx.shape[0]
    D = table.shape[1]
    nchunks = M // _GCH
    assert M % _GCH == 0

    def body(table_hbm, idx_hbm, out_hbm, idx_v, rows_v, sem):
        w = lax.axis_index("s") * 2 + lax.axis_index("c")
        nw = (nchunks - w + _NW - 1) // _NW

        def step(i, carry):
            base = (w + i * _NW) * _GCH
            pltpu.sync_copy(idx_hbm.at[pl.ds(base, _GCH)], idx_v)
            pltpu.async_copy(table_hbm.at[idx_v], rows_v, sem).wait()
            pltpu.sync_copy(rows_v, out_hbm.at[pl.ds(base, _GCH)])
            return carry

        lax.fori_loop(0, nw, step, 0)

    return pl.kernel(
        body,
        out_type=jax.ShapeDtypeStruct((M, D), jnp.float32),
        mesh=_SC_MESH,
        compiler_params=pltpu.CompilerParams(use_tc_tiling_on_sc=False),
        scratch_types=[pltpu.VMEM((_GCH,), jnp.int32),
                       pltpu.VMEM((_GCH, D), jnp.float32),
                       pltpu.SemaphoreType.DMA],
    )(table, idx)

EB = 1280
TB = 1280
NB = 2000


def _silu(x):
    return x * (1.0 / (1.0 + jnp.exp(-x)))


def _row_spec(rb, cols):
    return pl.BlockSpec((rb, cols), lambda i: (i, 0))


def _full_spec(shape):
    nd = len(shape)
    return pl.BlockSpec(shape, lambda i: (0,) * nd)


def _mkt_body(m_ref, rbf_ref, wmkt_ref, wrbf3_ref, wdown_ref, out_ref):
    t = _silu(jnp.dot(m_ref[...], wmkt_ref[...], preferred_element_type=jnp.float32))
    t = t * jnp.dot(rbf_ref[...], wrbf3_ref[...], preferred_element_type=jnp.float32)
    out_ref[...] = _silu(jnp.dot(t, wdown_ref[...], preferred_element_type=jnp.float32))


def _stage_mkt(m_st, rbf, W_mkt, W_rbf3, W_down):
    E, D = m_st.shape
    K = W_down.shape[1]
    return pl.pallas_call(
        _mkt_body,
        grid=(E // EB,),
        in_specs=[_row_spec(EB, D), _row_spec(EB, rbf.shape[1]),
                  _full_spec(W_mkt.shape), _full_spec(W_rbf3.shape), _full_spec(W_down.shape)],
        out_specs=_row_spec(EB, K),
        out_shape=jax.ShapeDtypeStruct((E, K), jnp.float32),
    )(m_st, rbf, W_mkt, W_rbf3, W_down)


def _bilinear_body(mg_ref, cbf_ref, s_ref, srep_ref, wb_ref, out_ref):
    cbf_exp = jnp.dot(cbf_ref[...], s_ref[...], preferred_element_type=jnp.float32)
    mg_exp = jnp.dot(mg_ref[...], srep_ref[...], preferred_element_type=jnp.float32)
    out_ref[...] = jnp.dot(cbf_exp * mg_exp, wb_ref[...], preferred_element_type=jnp.float32)


def _stage_bilinear(m_kt_g, cbf, S, Srep, Wb2):
    T, K = m_kt_g.shape
    B = Wb2.shape[1]
    return pl.pallas_call(
        _bilinear_body,
        grid=(T // TB,),
        in_specs=[_row_spec(TB, K), _row_spec(TB, cbf.shape[1]),
                  _full_spec(S.shape), _full_spec(Srep.shape), _full_spec(Wb2.shape)],
        out_specs=_row_spec(TB, B),
        out_shape=jax.ShapeDtypeStruct((T, B), jnp.float32),
    )(m_kt_g, cbf, S, Srep, Wb2)


def _xts_body(x_ref, w_ref, out_ref):
    out_ref[...] = _silu(jnp.dot(x_ref[...], w_ref[...], preferred_element_type=jnp.float32))


def _stage_xts(x, W_up_ts):
    E, B = x.shape
    D = W_up_ts.shape[1]
    return pl.pallas_call(
        _xts_body,
        grid=(E // EB,),
        in_specs=[_row_spec(EB, B), _full_spec(W_up_ts.shape)],
        out_specs=_row_spec(EB, D),
        out_shape=jax.ShapeDtypeStruct((E, D), jnp.float32),
    )(x, W_up_ts)


def _edge_chain_body(m_ref, rbf_ref, x_ref, xts_ref,
                     wskip_ref, wupst_ref, wrb0_ref, wrb1_ref, wra0_ref, wra1_ref,
                     waerbf_ref, m_out_ref, xa_pre_ref):
    m_st = m_ref[...]
    x_skip = jnp.dot(m_st, wskip_ref[...], preferred_element_type=jnp.float32)
    x_st = _silu(jnp.dot(x_ref[...], wupst_ref[...], preferred_element_type=jnp.float32))
    x3 = (x_st + xts_ref[...]) * INV_SQRT_2
    x = (x_skip + x3) * INV_SQRT_2
    y = _silu(jnp.dot(x, wrb0_ref[...], preferred_element_type=jnp.float32))
    y = _silu(jnp.dot(y, wrb1_ref[...], preferred_element_type=jnp.float32))
    x = (x + y) * INV_SQRT_2
    m = (m_st + x) * INV_SQRT_2
    y = _silu(jnp.dot(m, wra0_ref[...], preferred_element_type=jnp.float32))
    y = _silu(jnp.dot(y, wra1_ref[...], preferred_element_type=jnp.float32))
    m = (m + y) * INV_SQRT_2
    m_out_ref[...] = m
    xa_pre_ref[...] = m * jnp.dot(rbf_ref[...], waerbf_ref[...], preferred_element_type=jnp.float32)


def _stage_edge_chain(m_st, rbf, x, xts_sw, W_skip, W_up_st, Wrb0, Wrb1, Wra0, Wra1, W_ae_rbf):
    E, D = m_st.shape
    return pl.pallas_call(
        _edge_chain_body,
        grid=(E // EB,),
        in_specs=[_row_spec(EB, D), _row_spec(EB, rbf.shape[1]), _row_spec(EB, x.shape[1]),
                  _row_spec(EB, D)] + [_full_spec(w.shape) for w in
                                       (W_skip, W_up_st, Wrb0, Wrb1, Wra0, Wra1, W_ae_rbf)],
        out_specs=[_row_spec(EB, D), _row_spec(EB, D)],
        out_shape=[jax.ShapeDtypeStruct((E, D), jnp.float32),
                   jax.ShapeDtypeStruct((E, D), jnp.float32)],
    )(m_st, rbf, x, xts_sw, W_skip, W_up_st, Wrb0, Wrb1, Wra0, Wra1, W_ae_rbf)


def _node_body(xa_ref, h_ref, wd_ref, wa0_ref, wa1_ref, out_ref):
    xa = jnp.sum(xa_ref[...], axis=0)
    xa = _silu(jnp.dot(xa, wd_ref[...], preferred_element_type=jnp.float32))
    y = _silu(jnp.dot(xa, wa0_ref[...], preferred_element_type=jnp.float32))
    y = _silu(jnp.dot(y, wa1_ref[...], preferred_element_type=jnp.float32))
    xa = (xa + y) * INV_SQRT_2
    out_ref[...] = (h_ref[...] + xa) * INV_SQRT_2


def _stage_node(xa_parts, h, W_ae_dense, Wa0, Wa1):
    P, N, D = xa_parts.shape
    A = h.shape[1]
    return pl.pallas_call(
        _node_body,
        grid=(N // NB,),
        in_specs=[pl.BlockSpec((P, NB, D), lambda i: (0, i, 0)), _row_spec(NB, A),
                  _full_spec(W_ae_dense.shape), _full_spec(Wa0.shape), _full_spec(Wa1.shape)],
        out_specs=_row_spec(NB, A),
        out_shape=jax.ShapeDtypeStruct((N, A), jnp.float32),
    )(xa_parts, h, W_ae_dense, Wa0, Wa1)


def _final_body(hs_ref, ht_ref, m_ref, ws0_ref, ws1_ref, ws2_ref, wm0_ref, wm1_ref, out_ref):
    m = m_ref[...]
    m2 = _silu(jnp.dot(hs_ref[...], ws0_ref[...], preferred_element_type=jnp.float32)
               + jnp.dot(ht_ref[...], ws1_ref[...], preferred_element_type=jnp.float32)
               + jnp.dot(m, ws2_ref[...], preferred_element_type=jnp.float32))
    y = _silu(jnp.dot(m2, wm0_ref[...], preferred_element_type=jnp.float32))
    y = _silu(jnp.dot(y, wm1_ref[...], preferred_element_type=jnp.float32))
    m2 = (m2 + y) * INV_SQRT_2
    out_ref[...] = (m + m2) * INV_SQRT_2


def _stage_final(hs, ht, m, Ws0, Ws1, Ws2, Wm0, Wm1):
    E, D = m.shape
    return pl.pallas_call(
        _final_body,
        grid=(E // EB,),
        in_specs=[_row_spec(EB, hs.shape[1]), _row_spec(EB, ht.shape[1]), _row_spec(EB, D)]
                 + [_full_spec(w.shape) for w in (Ws0, Ws1, Ws2, Wm0, Wm1)],
        out_specs=_row_spec(EB, D),
        out_shape=jax.ShapeDtypeStruct((E, D), jnp.float32),
    )(hs, ht, m, Ws0, Ws1, Ws2, Wm0, Wm1)


def kernel(h, m_st, rbf, cbf, idx_s, idx_t, idx_swap, id3_kt, id3_st, id3_ragged_idx,
           W_skip, W_mkt, W_rbf3, W_down, W_bil, W_up_st, W_up_ts,
           W_res_before, W_res_after, W_ae_rbf, W_ae_dense, W_ae_res, W_self, W_res_m):
    E = m_st.shape[0]
    N, A = h.shape
    D = m_st.shape[1]
    C, K, B = W_bil.shape

    Wb2 = W_bil.reshape(C * K, B)
    S = jnp.kron(jnp.eye(C, dtype=jnp.float32), jnp.ones((1, K), jnp.float32))
    Srep = jnp.kron(jnp.ones((1, C), jnp.float32), jnp.eye(K, dtype=jnp.float32))
    Wrb0, Wrb1 = W_res_before[0, 0], W_res_before[0, 1]
    Wra0, Wra1 = W_res_after[0, 0], W_res_after[0, 1]
    Wa0, Wa1 = W_ae_res[0, 0], W_ae_res[0, 1]
    Ws0, Ws1, Ws2 = W_self[:A], W_self[A:2 * A], W_self[2 * A:]
    Wm0, Wm1 = W_res_m[0, 0], W_res_m[0, 1]

    id3_kt = id3_kt.astype(jnp.int32)
    idx_swap = idx_swap.astype(jnp.int32)
    idx_s = idx_s.astype(jnp.int32)
    idx_t = idx_t.astype(jnp.int32)

    m_kt = _stage_mkt(m_st, rbf, W_mkt, W_rbf3, W_down)
    m_kt_g = _sc_gather(m_kt, id3_kt)
    v = _stage_bilinear(m_kt_g, cbf, S, Srep, Wb2)
    x = jax.ops.segment_sum(v, id3_st, num_segments=E)
    x_ts = _stage_xts(x, W_up_ts)
    xts_sw = _sc_gather(x_ts, idx_swap)
    m_mid, xa_pre = _stage_edge_chain(m_st, rbf, x, xts_sw,
                                      W_skip, W_up_st, Wrb0, Wrb1, Wra0, Wra1, W_ae_rbf)
    xa = jax.ops.segment_sum(xa_pre, idx_t, num_segments=N)
    h_new = _stage_node(xa[None], h, W_ae_dense, Wa0, Wa1)
    hs = _sc_gather(h_new, idx_s)
    ht = _sc_gather(h_new, idx_t)
    m_new = _stage_final(hs, ht, m_mid, Ws0, Ws1, Ws2, Wm0, Wm1)
    return h_new, m_new


# speedup vs baseline: 3.6342x; 1.4799x over previous
import functools

import jax
import jax.numpy as jnp
from jax import lax
from jax.experimental import pallas as pl
from jax.experimental.pallas import tpu as pltpu
from jax.experimental.pallas import tpu_sc as plsc

INV_SQRT_2 = 0.7071067811865475

_SC_MESH = plsc.VectorSubcoreMesh(core_axis_name="c", subcore_axis_name="s")
_NW = 32
_GCH = 128


def _sc_gather(table, idx):
    M = idx.shape[0]
    D = table.shape[1]
    nchunks = M // _GCH
    assert M % _GCH == 0

    def body(table_hbm, idx_hbm, out_hbm, idx_v, rows_v, sem):
        w = lax.axis_index("s") * 2 + lax.axis_index("c")
        nw = (nchunks - w + _NW - 1) // _NW

        def step(i, carry):
            base = (w + i * _NW) * _GCH
            pltpu.sync_copy(idx_hbm.at[pl.ds(base, _GCH)], idx_v)
            pltpu.async_copy(table_hbm.at[idx_v], rows_v, sem).wait()
            pltpu.sync_copy(rows_v, out_hbm.at[pl.ds(base, _GCH)])
            return carry

        lax.fori_loop(0, nw, step, 0)

    return pl.kernel(
        body,
        out_type=jax.ShapeDtypeStruct((M, D), jnp.float32),
        mesh=_SC_MESH,
        compiler_params=pltpu.CompilerParams(use_tc_tiling_on_sc=False),
        scratch_types=[pltpu.VMEM((_GCH,), jnp.int32),
                       pltpu.VMEM((_GCH, D), jnp.float32),
                       pltpu.SemaphoreType.DMA],
    )(table, idx)

EB = 1280
TB = 1280
NB = 2000


def _silu(x):
    return x * (1.0 / (1.0 + jnp.exp(-x)))


def _row_spec(rb, cols):
    return pl.BlockSpec((rb, cols), lambda i: (i, 0))


def _full_spec(shape):
    nd = len(shape)
    return pl.BlockSpec(shape, lambda i: (0,) * nd)


def _sc_scatter_sum(vals, idx, zeros, N):
    E, D = vals.shape
    nchunks = E // _GCH
    half = nchunks // 2
    NS = 16
    n_per_sub = N // NS

    def body(vals_hbm, idx_hbm, zeros_hbm, out_hbm, idx_v, rows_v, acc, sem):
        c = lax.axis_index("c")
        s = lax.axis_index("s")
        pltpu.sync_copy(zeros_hbm, acc.at[pl.ds(s * n_per_sub, n_per_sub)])
        plsc.subcore_barrier()

        nw = (half - s + NS - 1) // NS

        def step(k, carry):
            base = (c * half + s + k * NS) * _GCH
            pltpu.sync_copy(idx_hbm.at[pl.ds(base, _GCH)], idx_v)
            pltpu.sync_copy(vals_hbm.at[pl.ds(base, _GCH)], rows_v)
            pltpu.sync_copy(rows_v, acc.at[idx_v], add=True)
            return carry

        lax.fori_loop(0, nw, step, 0)
        plsc.subcore_barrier()
        pltpu.sync_copy(acc.at[pl.ds(s * n_per_sub, n_per_sub)],
                        out_hbm.at[c, pl.ds(s * n_per_sub, n_per_sub)])

    return pl.kernel(
        body,
        out_type=jax.ShapeDtypeStruct((2, N, D), jnp.float32),
        mesh=_SC_MESH,
        compiler_params=pltpu.CompilerParams(use_tc_tiling_on_sc=False),
        scratch_types=[pltpu.VMEM((_GCH,), jnp.int32),
                       pltpu.VMEM((_GCH, D), jnp.float32),
                       pltpu.VMEM_SHARED((N, D), jnp.float32),
                       pltpu.SemaphoreType.DMA],
    )(vals, idx, zeros)


def _sc_segsum_sorted(vals, idx, tb, zeros, E):
    T, D = vals.shape
    NS = 16
    NQ = 4
    Q = E // NQ
    QP = Q + 16
    zlen = QP // NS

    def body(vals_hbm, idx_hbm, tb_hbm, zeros_hbm, out_hbm, tb_v, idx_raw, idx_v, rows_v, acc, sem):
        c = lax.axis_index("c")
        s = lax.axis_index("s")
        pltpu.sync_copy(tb_hbm, tb_v)
        iota = lax.iota(jnp.int32, 16)
        tvec = tb_v[...]

        for j in range(2):
            t0 = jnp.where(c == 0, tvec[j], tvec[2 + j])
            t1 = jnp.where(c == 0, tvec[j + 1], tvec[3 + j])
            e0 = (c * 2 + j) * Q
            pltpu.sync_copy(zeros_hbm, acc.at[pl.ds(s * zlen, zlen)])
            plsc.subcore_barrier()

            t0a = (t0 // 8) * 8
            n = (t1 - t0a + _GCH - 1) // _GCH
            nw = (n - s + NS - 1) // NS

            def step(k, carry, t0=t0, t1=t1, e0=e0, t0a=t0a):
                nominal = t0a + (s + k * NS) * _GCH
                start = jnp.minimum(nominal, T - _GCH)
                pltpu.sync_copy(idx_hbm.at[pl.ds(start, _GCH)], idx_raw)
                pltpu.sync_copy(vals_hbm.at[pl.ds(start, _GCH)], rows_v)
                lo = jnp.maximum(t0, nominal)
                for i in range(_GCH // 16):
                    ids = idx_raw[pl.ds(i * 16, 16)]
                    tpos = start + i * 16 + iota
                    mask = (tpos >= lo) & (tpos < t1)
                    idx_v[pl.ds(i * 16, 16)] = jnp.where(mask, ids - e0, Q)
                pltpu.sync_copy(rows_v, acc.at[idx_v], add=True)
                return carry

            lax.fori_loop(0, nw, step, 0)
            plsc.subcore_barrier()
            base = s * 2496
            if_last = s == NS - 1

            @pl.when(if_last)
            def _():
                pltpu.sync_copy(acc.at[pl.ds(base, Q - 15 * 2496)],
                                out_hbm.at[pl.ds(e0 + base, Q - 15 * 2496)])

            @pl.when(jnp.logical_not(if_last))
            def _():
                pltpu.sync_copy(acc.at[pl.ds(base, 2496)],
                                out_hbm.at[pl.ds(e0 + base, 2496)])
            plsc.subcore_barrier()

    return pl.kernel(
        body,
        out_type=jax.ShapeDtypeStruct((E, D), jnp.float32),
        mesh=_SC_MESH,
        compiler_params=pltpu.CompilerParams(use_tc_tiling_on_sc=False),
        scratch_types=[pltpu.VMEM((16,), jnp.int32),
                       pltpu.VMEM((_GCH,), jnp.int32),
                       pltpu.VMEM((_GCH,), jnp.int32),
                       pltpu.VMEM((_GCH, D), jnp.float32),
                       pltpu.VMEM_SHARED((QP, D), jnp.float32),
                       pltpu.SemaphoreType.DMA],
    )(vals, idx, tb, zeros)


def _mkt_body(m_ref, rbf_ref, wmkt_ref, wrbf3_ref, wdown_ref, out_ref):
    t = _silu(jnp.dot(m_ref[...], wmkt_ref[...], preferred_element_type=jnp.float32))
    t = t * jnp.dot(rbf_ref[...], wrbf3_ref[...], preferred_element_type=jnp.float32)
    out_ref[...] = _silu(jnp.dot(t, wdown_ref[...], preferred_element_type=jnp.float32))


def _stage_mkt(m_st, rbf, W_mkt, W_rbf3, W_down):
    E, D = m_st.shape
    K = W_down.shape[1]
    return pl.pallas_call(
        _mkt_body,
        grid=(E // EB,),
        in_specs=[_row_spec(EB, D), _row_spec(EB, rbf.shape[1]),
                  _full_spec(W_mkt.shape), _full_spec(W_rbf3.shape), _full_spec(W_down.shape)],
        out_specs=_row_spec(EB, K),
        out_shape=jax.ShapeDtypeStruct((E, K), jnp.float32),
    )(m_st, rbf, W_mkt, W_rbf3, W_down)


def _bilinear_body(mg_ref, cbf_ref, s_ref, srep_ref, wb_ref, out_ref):
    cbf_exp = jnp.dot(cbf_ref[...], s_ref[...], preferred_element_type=jnp.float32)
    mg_exp = jnp.dot(mg_ref[...], srep_ref[...], preferred_element_type=jnp.float32)
    out_ref[...] = jnp.dot(cbf_exp * mg_exp, wb_ref[...], preferred_element_type=jnp.float32)


def _stage_bilinear(m_kt_g, cbf, S, Srep, Wb2):
    T, K = m_kt_g.shape
    B = Wb2.shape[1]
    return pl.pallas_call(
        _bilinear_body,
        grid=(T // TB,),
        in_specs=[_row_spec(TB, K), _row_spec(TB, cbf.shape[1]),
                  _full_spec(S.shape), _full_spec(Srep.shape), _full_spec(Wb2.shape)],
        out_specs=_row_spec(TB, B),
        out_shape=jax.ShapeDtypeStruct((T, B), jnp.float32),
    )(m_kt_g, cbf, S, Srep, Wb2)


def _xts_body(x_ref, w_ref, out_ref):
    out_ref[...] = _silu(jnp.dot(x_ref[...], w_ref[...], preferred_element_type=jnp.float32))


def _stage_xts(x, W_up_ts):
    E, B = x.shape
    D = W_up_ts.shape[1]
    return pl.pallas_call(
        _xts_body,
        grid=(E // EB,),
        in_specs=[_row_spec(EB, B), _full_spec(W_up_ts.shape)],
        out_specs=_row_spec(EB, D),
        out_shape=jax.ShapeDtypeStruct((E, D), jnp.float32),
    )(x, W_up_ts)


def _edge_chain_body(m_ref, rbf_ref, x_ref, xts_ref,
                     wskip_ref, wupst_ref, wrb0_ref, wrb1_ref, wra0_ref, wra1_ref,
                     waerbf_ref, m_out_ref, xa_pre_ref):
    m_st = m_ref[...]
    x_skip = jnp.dot(m_st, wskip_ref[...], preferred_element_type=jnp.float32)
    x_st = _silu(jnp.dot(x_ref[...], wupst_ref[...], preferred_element_type=jnp.float32))
    x3 = (x_st + xts_ref[...]) * INV_SQRT_2
    x = (x_skip + x3) * INV_SQRT_2
    y = _silu(jnp.dot(x, wrb0_ref[...], preferred_element_type=jnp.float32))
    y = _silu(jnp.dot(y, wrb1_ref[...], preferred_element_type=jnp.float32))
    x = (x + y) * INV_SQRT_2
    m = (m_st + x) * INV_SQRT_2
    y = _silu(jnp.dot(m, wra0_ref[...], preferred_element_type=jnp.float32))
    y = _silu(jnp.dot(y, wra1_ref[...], preferred_element_type=jnp.float32))
    m = (m + y) * INV_SQRT_2
    m_out_ref[...] = m
    xa_pre_ref[...] = m * jnp.dot(rbf_ref[...], waerbf_ref[...], preferred_element_type=jnp.float32)


def _stage_edge_chain(m_st, rbf, x, xts_sw, W_skip, W_up_st, Wrb0, Wrb1, Wra0, Wra1, W_ae_rbf):
    E, D = m_st.shape
    return pl.pallas_call(
        _edge_chain_body,
        grid=(E // EB,),
        in_specs=[_row_spec(EB, D), _row_spec(EB, rbf.shape[1]), _row_spec(EB, x.shape[1]),
                  _row_spec(EB, D)] + [_full_spec(w.shape) for w in
                                       (W_skip, W_up_st, Wrb0, Wrb1, Wra0, Wra1, W_ae_rbf)],
        out_specs=[_row_spec(EB, D), _row_spec(EB, D)],
        out_shape=[jax.ShapeDtypeStruct((E, D), jnp.float32),
                   jax.ShapeDtypeStruct((E, D), jnp.float32)],
    )(m_st, rbf, x, xts_sw, W_skip, W_up_st, Wrb0, Wrb1, Wra0, Wra1, W_ae_rbf)


def _node_body(xa_ref, h_ref, wd_ref, wa0_ref, wa1_ref, out_ref):
    xa = jnp.sum(xa_ref[...], axis=0)
    xa = _silu(jnp.dot(xa, wd_ref[...], preferred_element_type=jnp.float32))
    y = _silu(jnp.dot(xa, wa0_ref[...], preferred_element_type=jnp.float32))
    y = _silu(jnp.dot(y, wa1_ref[...], preferred_element_type=jnp.float32))
    xa = (xa + y) * INV_SQRT_2
    out_ref[...] = (h_ref[...] + xa) * INV_SQRT_2


def _stage_node(xa_parts, h, W_ae_dense, Wa0, Wa1):
    P, N, D = xa_parts.shape
    A = h.shape[1]
    return pl.pallas_call(
        _node_body,
        grid=(N // NB,),
        in_specs=[pl.BlockSpec((P, NB, D), lambda i: (0, i, 0)), _row_spec(NB, A),
                  _full_spec(W_ae_dense.shape), _full_spec(Wa0.shape), _full_spec(Wa1.shape)],
        out_specs=_row_spec(NB, A),
        out_shape=jax.ShapeDtypeStruct((N, A), jnp.float32),
    )(xa_parts, h, W_ae_dense, Wa0, Wa1)


def _final_body(hs_ref, ht_ref, m_ref, ws0_ref, ws1_ref, ws2_ref, wm0_ref, wm1_ref, out_ref):
    m = m_ref[...]
    m2 = _silu(jnp.dot(hs_ref[...], ws0_ref[...], preferred_element_type=jnp.float32)
               + jnp.dot(ht_ref[...], ws1_ref[...], preferred_element_type=jnp.float32)
               + jnp.dot(m, ws2_ref[...], preferred_element_type=jnp.float32))
    y = _silu(jnp.dot(m2, wm0_ref[...], preferred_element_type=jnp.float32))
    y = _silu(jnp.dot(y, wm1_ref[...], preferred_element_type=jnp.float32))
    m2 = (m2 + y) * INV_SQRT_2
    out_ref[...] = (m + m2) * INV_SQRT_2


def _stage_final(hs, ht, m, Ws0, Ws1, Ws2, Wm0, Wm1):
    E, D = m.shape
    return pl.pallas_call(
        _final_body,
        grid=(E // EB,),
        in_specs=[_row_spec(EB, hs.shape[1]), _row_spec(EB, ht.shape[1]), _row_spec(EB, D)]
                 + [_full_spec(w.shape) for w in (Ws0, Ws1, Ws2, Wm0, Wm1)],
        out_specs=_row_spec(EB, D),
        out_shape=jax.ShapeDtypeStruct((E, D), jnp.float32),
    )(hs, ht, m, Ws0, Ws1, Ws2, Wm0, Wm1)


def kernel(h, m_st, rbf, cbf, idx_s, idx_t, idx_swap, id3_kt, id3_st, id3_ragged_idx,
           W_skip, W_mkt, W_rbf3, W_down, W_bil, W_up_st, W_up_ts,
           W_res_before, W_res_after, W_ae_rbf, W_ae_dense, W_ae_res, W_self, W_res_m):
    E = m_st.shape[0]
    N, A = h.shape
    D = m_st.shape[1]
    C, K, B = W_bil.shape

    Wb2 = W_bil.reshape(C * K, B)
    S = jnp.kron(jnp.eye(C, dtype=jnp.float32), jnp.ones((1, K), jnp.float32))
    Srep = jnp.kron(jnp.ones((1, C), jnp.float32), jnp.eye(K, dtype=jnp.float32))
    Wrb0, Wrb1 = W_res_before[0, 0], W_res_before[0, 1]
    Wra0, Wra1 = W_res_after[0, 0], W_res_after[0, 1]
    Wa0, Wa1 = W_ae_res[0, 0], W_ae_res[0, 1]
    Ws0, Ws1, Ws2 = W_self[:A], W_self[A:2 * A], W_self[2 * A:]
    Wm0, Wm1 = W_res_m[0, 0], W_res_m[0, 1]

    id3_kt = id3_kt.astype(jnp.int32)
    idx_swap = idx_swap.astype(jnp.int32)
    idx_s = idx_s.astype(jnp.int32)
    idx_t = idx_t.astype(jnp.int32)

    id3_st = id3_st.astype(jnp.int32)
    Q = E // 4
    tb = jnp.searchsorted(id3_st, jnp.arange(5, dtype=jnp.int32) * Q).astype(jnp.int32)
    tb = jnp.concatenate([tb, jnp.zeros((11,), jnp.int32)])
    zeros_n = jnp.zeros((N // 16, D), jnp.float32)
    zeros_q = jnp.zeros(((Q + 16) // 16, B), jnp.float32)

    m_kt = _stage_mkt(m_st, rbf, W_mkt, W_rbf3, W_down)
    m_kt_g = _sc_gather(m_kt, id3_kt)
    v = _stage_bilinear(m_kt_g, cbf, S, Srep, Wb2)
    x = _sc_segsum_sorted(v, id3_st, tb, zeros_q, E)
    x_ts = _stage_xts(x, W_up_ts)
    xts_sw = _sc_gather(x_ts, idx_swap)
    m_mid, xa_pre = _stage_edge_chain(m_st, rbf, x, xts_sw,
                                      W_skip, W_up_st, Wrb0, Wrb1, Wra0, Wra1, W_ae_rbf)
    xa_parts = _sc_scatter_sum(xa_pre, idx_t, zeros_n, N)
    h_new = _stage_node(xa_parts, h, W_ae_dense, Wa0, Wa1)
    hs = _sc_gather(h_new, idx_s)
    ht = _sc_gather(h_new, idx_t)
    m_new = _stage_final(hs, ht, m_mid, Ws0, Ws1, Ws2, Wm0, Wm1)
    return h_new, m_new


# speedup vs baseline: 4.0776x; 1.1220x over previous
import functools

import jax
import jax.numpy as jnp
from jax import lax
from jax.experimental import pallas as pl
from jax.experimental.pallas import tpu as pltpu
from jax.experimental.pallas import tpu_sc as plsc

INV_SQRT_2 = 0.7071067811865475

_SC_MESH = plsc.VectorSubcoreMesh(core_axis_name="c", subcore_axis_name="s")
_NW = 32
_GCH = 128


def _sc_gather(table, idx):
    M = idx.shape[0]
    D = table.shape[1]
    nchunks = M // _GCH
    assert M % _GCH == 0

    def body(table_hbm, idx_hbm, out_hbm, idx_v, rows_v, sem):
        w = lax.axis_index("s") * 2 + lax.axis_index("c")
        nw = (nchunks - w + _NW - 1) // _NW

        def step(i, carry):
            base = (w + i * _NW) * _GCH
            pltpu.sync_copy(idx_hbm.at[pl.ds(base, _GCH)], idx_v)
            pltpu.async_copy(table_hbm.at[idx_v], rows_v, sem).wait()
            pltpu.sync_copy(rows_v, out_hbm.at[pl.ds(base, _GCH)])
            return carry

        lax.fori_loop(0, nw, step, 0)

    return pl.kernel(
        body,
        out_type=jax.ShapeDtypeStruct((M, D), jnp.float32),
        mesh=_SC_MESH,
        compiler_params=pltpu.CompilerParams(use_tc_tiling_on_sc=False),
        scratch_types=[pltpu.VMEM((_GCH,), jnp.int32),
                       pltpu.VMEM((_GCH, D), jnp.float32),
                       pltpu.SemaphoreType.DMA],
    )(table, idx)

EB = 1280
TB = 1280
NB = 2000


def _silu(x):
    return x * (1.0 / (1.0 + jnp.exp(-x)))


def _row_spec(rb, cols):
    return pl.BlockSpec((rb, cols), lambda i: (i, 0))


def _full_spec(shape):
    nd = len(shape)
    return pl.BlockSpec(shape, lambda i: (0,) * nd)


def _sc_gather2(table0, idx0, table1, idx1):
    M = idx0.shape[0]
    D = table0.shape[1]
    nchunks = M // _GCH

    def body(t0_hbm, i0_hbm, t1_hbm, i1_hbm, out0_hbm, out1_hbm,
             idx_v0, idx_v1, rows_v0, rows_v1, sem0, sem1):
        w = lax.axis_index("s") * 2 + lax.axis_index("c")
        nw = (nchunks - w + _NW - 1) // _NW

        def step(i, carry):
            base = (w + i * _NW) * _GCH
            pltpu.sync_copy(i0_hbm.at[pl.ds(base, _GCH)], idx_v0)
            pltpu.sync_copy(i1_hbm.at[pl.ds(base, _GCH)], idx_v1)
            cp0 = pltpu.async_copy(t0_hbm.at[idx_v0], rows_v0, sem0)
            cp1 = pltpu.async_copy(t1_hbm.at[idx_v1], rows_v1, sem1)
            cp0.wait()
            pltpu.sync_copy(rows_v0, out0_hbm.at[pl.ds(base, _GCH)])
            cp1.wait()
            pltpu.sync_copy(rows_v1, out1_hbm.at[pl.ds(base, _GCH)])
            return carry

        lax.fori_loop(0, nw, step, 0)

    return pl.kernel(
        body,
        out_type=(jax.ShapeDtypeStruct((M, D), jnp.float32),
                  jax.ShapeDtypeStruct((M, D), jnp.float32)),
        mesh=_SC_MESH,
        compiler_params=pltpu.CompilerParams(use_tc_tiling_on_sc=False),
        scratch_types=[pltpu.VMEM((_GCH,), jnp.int32),
                       pltpu.VMEM((_GCH,), jnp.int32),
                       pltpu.VMEM((_GCH, D), jnp.float32),
                       pltpu.VMEM((_GCH, D), jnp.float32),
                       pltpu.SemaphoreType.DMA,
                       pltpu.SemaphoreType.DMA],
    )(table0, idx0, table1, idx1)


def _sc_scatter_sum(vals, idx, zeros, N):
    E, D = vals.shape
    nchunks = E // _GCH
    half = nchunks // 2
    NS = 16
    n_per_sub = N // NS

    def body(vals_hbm, idx_hbm, zeros_hbm, out_hbm, idx_v, rows_v, acc, sem):
        c = lax.axis_index("c")
        s = lax.axis_index("s")
        pltpu.sync_copy(zeros_hbm, acc.at[pl.ds(s * n_per_sub, n_per_sub)])
        plsc.subcore_barrier()

        nw = (half - s + NS - 1) // NS

        def step(k, carry):
            base = (c * half + s + k * NS) * _GCH
            pltpu.sync_copy(idx_hbm.at[pl.ds(base, _GCH)], idx_v)
            pltpu.sync_copy(vals_hbm.at[pl.ds(base, _GCH)], rows_v)
            pltpu.sync_copy(rows_v, acc.at[idx_v], add=True)
            return carry

        lax.fori_loop(0, nw, step, 0)
        plsc.subcore_barrier()
        pltpu.sync_copy(acc.at[pl.ds(s * n_per_sub, n_per_sub)],
                        out_hbm.at[c, pl.ds(s * n_per_sub, n_per_sub)])

    return pl.kernel(
        body,
        out_type=jax.ShapeDtypeStruct((2, N, D), jnp.float32),
        mesh=_SC_MESH,
        compiler_params=pltpu.CompilerParams(use_tc_tiling_on_sc=False),
        scratch_types=[pltpu.VMEM((_GCH,), jnp.int32),
                       pltpu.VMEM((_GCH, D), jnp.float32),
                       pltpu.VMEM_SHARED((N, D), jnp.float32),
                       pltpu.SemaphoreType.DMA],
    )(vals, idx, zeros)


def _sc_segsum_sorted(vals, idx, tb, zeros, E):
    T, D = vals.shape
    NS = 16
    NQ = 4
    Q = E // NQ
    QP = Q + 16
    zlen = QP // NS

    def body(vals_hbm, idx_hbm, tb_hbm, zeros_hbm, out_hbm, tb_v, idx_raw, idx_v, rows_v, acc, sem):
        c = lax.axis_index("c")
        s = lax.axis_index("s")
        pltpu.sync_copy(tb_hbm, tb_v)
        iota = lax.iota(jnp.int32, 16)
        tvec = tb_v[...]

        for j in range(2):
            t0 = jnp.where(c == 0, tvec[j], tvec[2 + j])
            t1 = jnp.where(c == 0, tvec[j + 1], tvec[3 + j])
            e0 = (c * 2 + j) * Q
            pltpu.sync_copy(zeros_hbm, acc.at[pl.ds(s * zlen, zlen)])
            plsc.subcore_barrier()

            t0a = (t0 // 8) * 8
            n = (t1 - t0a + _GCH - 1) // _GCH
            nw = (n - s + NS - 1) // NS

            def step(k, carry, t0=t0, t1=t1, e0=e0, t0a=t0a):
                nominal = t0a + (s + k * NS) * _GCH
                start = jnp.minimum(nominal, T - _GCH)
                pltpu.sync_copy(idx_hbm.at[pl.ds(start, _GCH)], idx_raw)
                pltpu.sync_copy(vals_hbm.at[pl.ds(start, _GCH)], rows_v)
                lo = jnp.maximum(t0, nominal)
                for i in range(_GCH // 16):
                    ids = idx_raw[pl.ds(i * 16, 16)]
                    tpos = start + i * 16 + iota
                    mask = (tpos >= lo) & (tpos < t1)
                    idx_v[pl.ds(i * 16, 16)] = jnp.where(mask, ids - e0, Q)
                pltpu.sync_copy(rows_v, acc.at[idx_v], add=True)
                return carry

            lax.fori_loop(0, nw, step, 0)
            plsc.subcore_barrier()
            base = s * 2496
            if_last = s == NS - 1

            @pl.when(if_last)
            def _():
                pltpu.sync_copy(acc.at[pl.ds(base, Q - 15 * 2496)],
                                out_hbm.at[pl.ds(e0 + base, Q - 15 * 2496)])

            @pl.when(jnp.logical_not(if_last))
            def _():
                pltpu.sync_copy(acc.at[pl.ds(base, 2496)],
                                out_hbm.at[pl.ds(e0 + base, 2496)])
            plsc.subcore_barrier()

    return pl.kernel(
        body,
        out_type=jax.ShapeDtypeStruct((E, D), jnp.float32),
        mesh=_SC_MESH,
        compiler_params=pltpu.CompilerParams(use_tc_tiling_on_sc=False),
        scratch_types=[pltpu.VMEM((16,), jnp.int32),
                       pltpu.VMEM((_GCH,), jnp.int32),
                       pltpu.VMEM((_GCH,), jnp.int32),
                       pltpu.VMEM((_GCH, D), jnp.float32),
                       pltpu.VMEM_SHARED((QP, D), jnp.float32),
                       pltpu.SemaphoreType.DMA],
    )(vals, idx, tb, zeros)


def _mkt_body(m_ref, rbf_ref, wmkt_ref, wrbf3_ref, wdown_ref, out_ref):
    t = _silu(jnp.dot(m_ref[...], wmkt_ref[...], preferred_element_type=jnp.float32))
    t = t * jnp.dot(rbf_ref[...], wrbf3_ref[...], preferred_element_type=jnp.float32)
    out_ref[...] = _silu(jnp.dot(t, wdown_ref[...], preferred_element_type=jnp.float32))


def _stage_mkt(m_st, rbf, W_mkt, W_rbf3, W_down):
    E, D = m_st.shape
    K = W_down.shape[1]
    return pl.pallas_call(
        _mkt_body,
        grid=(E // EB,),
        in_specs=[_row_spec(EB, D), _row_spec(EB, rbf.shape[1]),
                  _full_spec(W_mkt.shape), _full_spec(W_rbf3.shape), _full_spec(W_down.shape)],
        out_specs=_row_spec(EB, K),
        out_shape=jax.ShapeDtypeStruct((E, K), jnp.float32),
    )(m_st, rbf, W_mkt, W_rbf3, W_down)


def _bilinear_body(mg_ref, cbf_ref, s_ref, srep_ref, wb_ref, out_ref):
    C = cbf_ref.shape[1]
    cbf_exp = jnp.dot(cbf_ref[...], s_ref[...], preferred_element_type=jnp.float32)
    mg_exp = jnp.concatenate([mg_ref[...]] * C, axis=1)
    out_ref[...] = jnp.dot(cbf_exp * mg_exp, wb_ref[...], preferred_element_type=jnp.float32)


def _stage_bilinear(m_kt_g, cbf, S, Srep, Wb2):
    T, K = m_kt_g.shape
    B = Wb2.shape[1]
    return pl.pallas_call(
        _bilinear_body,
        grid=(T // TB,),
        in_specs=[_row_spec(TB, K), _row_spec(TB, cbf.shape[1]),
                  _full_spec(S.shape), _full_spec(Srep.shape), _full_spec(Wb2.shape)],
        out_specs=_row_spec(TB, B),
        out_shape=jax.ShapeDtypeStruct((T, B), jnp.float32),
    )(m_kt_g, cbf, S, Srep, Wb2)


def _xts_body(x_ref, w_ref, out_ref):
    out_ref[...] = _silu(jnp.dot(x_ref[...], w_ref[...], preferred_element_type=jnp.float32))


def _stage_xts(x, W_up_ts):
    E, B = x.shape
    D = W_up_ts.shape[1]
    return pl.pallas_call(
        _xts_body,
        grid=(E // EB,),
        in_specs=[_row_spec(EB, B), _full_spec(W_up_ts.shape)],
        out_specs=_row_spec(EB, D),
        out_shape=jax.ShapeDtypeStruct((E, D), jnp.float32),
    )(x, W_up_ts)


def _edge_chain_body(m_ref, rbf_ref, x_ref, xsw_ref,
                     wskip_ref, wupst_ref, wupts_ref, wrb0_ref, wrb1_ref, wra0_ref, wra1_ref,
                     waerbf_ref, m_out_ref, xa_pre_ref):
    m_st = m_ref[...]
    x_skip = jnp.dot(m_st, wskip_ref[...], preferred_element_type=jnp.float32)
    x_st = _silu(jnp.dot(x_ref[...], wupst_ref[...], preferred_element_type=jnp.float32))
    x_ts_sw = _silu(jnp.dot(xsw_ref[...], wupts_ref[...], preferred_element_type=jnp.float32))
    x3 = (x_st + x_ts_sw) * INV_SQRT_2
    x = (x_skip + x3) * INV_SQRT_2
    y = _silu(jnp.dot(x, wrb0_ref[...], preferred_element_type=jnp.float32))
    y = _silu(jnp.dot(y, wrb1_ref[...], preferred_element_type=jnp.float32))
    x = (x + y) * INV_SQRT_2
    m = (m_st + x) * INV_SQRT_2
    y = _silu(jnp.dot(m, wra0_ref[...], preferred_element_type=jnp.float32))
    y = _silu(jnp.dot(y, wra1_ref[...], preferred_element_type=jnp.float32))
    m = (m + y) * INV_SQRT_2
    m_out_ref[...] = m
    xa_pre_ref[...] = m * jnp.dot(rbf_ref[...], waerbf_ref[...], preferred_element_type=jnp.float32)


def _stage_edge_chain(m_st, rbf, x, x_sw, W_skip, W_up_st, W_up_ts, Wrb0, Wrb1, Wra0, Wra1, W_ae_rbf):
    E, D = m_st.shape
    return pl.pallas_call(
        _edge_chain_body,
        grid=(E // EB,),
        in_specs=[_row_spec(EB, D), _row_spec(EB, rbf.shape[1]), _row_spec(EB, x.shape[1]),
                  _row_spec(EB, x_sw.shape[1])] + [_full_spec(w.shape) for w in
                                       (W_skip, W_up_st, W_up_ts, Wrb0, Wrb1, Wra0, Wra1, W_ae_rbf)],
        out_specs=[_row_spec(EB, D), _row_spec(EB, D)],
        out_shape=[jax.ShapeDtypeStruct((E, D), jnp.float32),
                   jax.ShapeDtypeStruct((E, D), jnp.float32)],
    )(m_st, rbf, x, x_sw, W_skip, W_up_st, W_up_ts, Wrb0, Wrb1, Wra0, Wra1, W_ae_rbf)


def _node_body(xa_ref, h_ref, wd_ref, wa0_ref, wa1_ref, ws0_ref, ws1_ref,
               out_ref, hp_s_ref, hp_t_ref):
    xa = jnp.sum(xa_ref[...], axis=0)
    xa = _silu(jnp.dot(xa, wd_ref[...], preferred_element_type=jnp.float32))
    y = _silu(jnp.dot(xa, wa0_ref[...], preferred_element_type=jnp.float32))
    y = _silu(jnp.dot(y, wa1_ref[...], preferred_element_type=jnp.float32))
    xa = (xa + y) * INV_SQRT_2
    h_new = (h_ref[...] + xa) * INV_SQRT_2
    out_ref[...] = h_new
    hp_s_ref[...] = jnp.dot(h_new, ws0_ref[...], preferred_element_type=jnp.float32)
    hp_t_ref[...] = jnp.dot(h_new, ws1_ref[...], preferred_element_type=jnp.float32)


def _stage_node(xa_parts, h, W_ae_dense, Wa0, Wa1, Ws0, Ws1):
    P, N, D = xa_parts.shape
    A = h.shape[1]
    return pl.pallas_call(
        _node_body,
        grid=(N // NB,),
        in_specs=[pl.BlockSpec((P, NB, D), lambda i: (0, i, 0)), _row_spec(NB, A),
                  _full_spec(W_ae_dense.shape), _full_spec(Wa0.shape), _full_spec(Wa1.shape),
                  _full_spec(Ws0.shape), _full_spec(Ws1.shape)],
        out_specs=[_row_spec(NB, A), _row_spec(NB, D), _row_spec(NB, D)],
        out_shape=[jax.ShapeDtypeStruct((N, A), jnp.float32),
                   jax.ShapeDtypeStruct((N, D), jnp.float32),
                   jax.ShapeDtypeStruct((N, D), jnp.float32)],
    )(xa_parts, h, W_ae_dense, Wa0, Wa1, Ws0, Ws1)


def _final_body(hsp_ref, htp_ref, m_ref, ws2_ref, wm0_ref, wm1_ref, out_ref):
    m = m_ref[...]
    m2 = _silu(hsp_ref[...] + htp_ref[...]
               + jnp.dot(m, ws2_ref[...], preferred_element_type=jnp.float32))
    y = _silu(jnp.dot(m2, wm0_ref[...], preferred_element_type=jnp.float32))
    y = _silu(jnp.dot(y, wm1_ref[...], preferred_element_type=jnp.float32))
    m2 = (m2 + y) * INV_SQRT_2
    out_ref[...] = (m + m2) * INV_SQRT_2


def _stage_final(hsp, htp, m, Ws2, Wm0, Wm1):
    E, D = m.shape
    return pl.pallas_call(
        _final_body,
        grid=(E // EB,),
        in_specs=[_row_spec(EB, hsp.shape[1]), _row_spec(EB, htp.shape[1]), _row_spec(EB, D)]
                 + [_full_spec(w.shape) for w in (Ws2, Wm0, Wm1)],
        out_specs=_row_spec(EB, D),
        out_shape=jax.ShapeDtypeStruct((E, D), jnp.float32),
    )(hsp, htp, m, Ws2, Wm0, Wm1)


def kernel(h, m_st, rbf, cbf, idx_s, idx_t, idx_swap, id3_kt, id3_st, id3_ragged_idx,
           W_skip, W_mkt, W_rbf3, W_down, W_bil, W_up_st, W_up_ts,
           W_res_before, W_res_after, W_ae_rbf, W_ae_dense, W_ae_res, W_self, W_res_m):
    E = m_st.shape[0]
    N, A = h.shape
    D = m_st.shape[1]
    C, K, B = W_bil.shape

    Wb2 = W_bil.reshape(C * K, B)
    S = jnp.kron(jnp.eye(C, dtype=jnp.float32), jnp.ones((1, K), jnp.float32))
    Srep = jnp.kron(jnp.ones((1, C), jnp.float32), jnp.eye(K, dtype=jnp.float32))
    Wrb0, Wrb1 = W_res_before[0, 0], W_res_before[0, 1]
    Wra0, Wra1 = W_res_after[0, 0], W_res_after[0, 1]
    Wa0, Wa1 = W_ae_res[0, 0], W_ae_res[0, 1]
    Ws0, Ws1, Ws2 = W_self[:A], W_self[A:2 * A], W_self[2 * A:]
    Wm0, Wm1 = W_res_m[0, 0], W_res_m[0, 1]

    id3_kt = id3_kt.astype(jnp.int32)
    idx_swap = idx_swap.astype(jnp.int32)
    idx_s = idx_s.astype(jnp.int32)
    idx_t = idx_t.astype(jnp.int32)

    id3_st = id3_st.astype(jnp.int32)
    Q = E // 4
    tb = jnp.searchsorted(id3_st, jnp.arange(5, dtype=jnp.int32) * Q).astype(jnp.int32)
    tb = jnp.concatenate([tb, jnp.zeros((11,), jnp.int32)])
    zeros_n = jnp.zeros((N // 16, D), jnp.float32)
    zeros_q = jnp.zeros(((Q + 16) // 16, B), jnp.float32)

    m_kt = _stage_mkt(m_st, rbf, W_mkt, W_rbf3, W_down)
    m_kt_g = _sc_gather(m_kt, id3_kt)
    v = _stage_bilinear(m_kt_g, cbf, S, Srep, Wb2)
    x = _sc_segsum_sorted(v, id3_st, tb, zeros_q, E)
    x_sw = _sc_gather(x, idx_swap)
    m_mid, xa_pre = _stage_edge_chain(m_st, rbf, x, x_sw,
                                      W_skip, W_up_st, W_up_ts, Wrb0, Wrb1, Wra0, Wra1, W_ae_rbf)
    xa_parts = _sc_scatter_sum(xa_pre, idx_t, zeros_n, N)
    h_new, hp_s, hp_t = _stage_node(xa_parts, h, W_ae_dense, Wa0, Wa1, Ws0, Ws1)
    hsp, htp = _sc_gather2(hp_s, idx_s, hp_t, idx_t)
    m_new = _stage_final(hsp, htp, m_mid, Ws2, Wm0, Wm1)
    return h_new, m_new


# speedup vs baseline: 4.2099x; 1.0324x over previous
import functools

import jax
import jax.numpy as jnp
from jax import lax
from jax.experimental import pallas as pl
from jax.experimental.pallas import tpu as pltpu
from jax.experimental.pallas import tpu_sc as plsc

INV_SQRT_2 = 0.7071067811865475

_SC_MESH = plsc.VectorSubcoreMesh(core_axis_name="c", subcore_axis_name="s")
_NW = 32
_GCH = 128


def _sc_gather(table, idx, k):
    M = idx.shape[0] * _GCH
    D = table.shape[1]
    GB = _GCH * k
    nchunks = M // GB
    assert M % GB == 0

    def body(table_hbm, idx_hbm, out_hbm, idx_v, rows_v, sem):
        w = lax.axis_index("s") * 2 + lax.axis_index("c")
        nw = (nchunks - w + _NW - 1) // _NW

        def step(i, carry):
            c = w + i * _NW
            base = c * GB
            pltpu.sync_copy(idx_hbm.at[pl.ds(c * k, k)], idx_v)
            cps = [pltpu.async_copy(table_hbm.at[idx_v.at[j]],
                                    rows_v.at[pl.ds(j * _GCH, _GCH)], sem)
                   for j in range(k)]
            for cp in cps:
                cp.wait()
            pltpu.sync_copy(rows_v, out_hbm.at[pl.ds(base, GB)])
            return carry

        lax.fori_loop(0, nw, step, 0)

    return pl.kernel(
        body,
        out_type=jax.ShapeDtypeStruct((M, D), jnp.float32),
        mesh=_SC_MESH,
        compiler_params=pltpu.CompilerParams(use_tc_tiling_on_sc=False),
        scratch_types=[pltpu.VMEM((k, _GCH), jnp.int32),
                       pltpu.VMEM((GB, D), jnp.float32),
                       pltpu.SemaphoreType.DMA],
    )(table, idx)

EB = 1280
TB = 1280
NB = 2000


def _silu(x):
    return x * (1.0 / (1.0 + jnp.exp(-x)))


def _row_spec(rb, cols):
    return pl.BlockSpec((rb, cols), lambda i: (i, 0))


def _full_spec(shape):
    nd = len(shape)
    return pl.BlockSpec(shape, lambda i: (0,) * nd)


def _sc_gather2(table0, idx0, table1, idx1, k=2):
    M = idx0.shape[0] * _GCH
    D = table0.shape[1]
    GB = _GCH * k
    nchunks = M // GB
    assert M % GB == 0

    def body(t0_hbm, i0_hbm, t1_hbm, i1_hbm, out0_hbm, out1_hbm,
             idx_v0, idx_v1, rows_v0, rows_v1, sem0, sem1):
        w = lax.axis_index("s") * 2 + lax.axis_index("c")
        nw = (nchunks - w + _NW - 1) // _NW

        def step(i, carry):
            c = w + i * _NW
            base = c * GB
            pltpu.sync_copy(i0_hbm.at[pl.ds(c * k, k)], idx_v0)
            pltpu.sync_copy(i1_hbm.at[pl.ds(c * k, k)], idx_v1)
            cp0 = [pltpu.async_copy(t0_hbm.at[idx_v0.at[j]],
                                    rows_v0.at[pl.ds(j * _GCH, _GCH)], sem0)
                   for j in range(k)]
            cp1 = [pltpu.async_copy(t1_hbm.at[idx_v1.at[j]],
                                    rows_v1.at[pl.ds(j * _GCH, _GCH)], sem1)
                   for j in range(k)]
            for cp in cp0:
                cp.wait()
            pltpu.sync_copy(rows_v0, out0_hbm.at[pl.ds(base, GB)])
            for cp in cp1:
                cp.wait()
            pltpu.sync_copy(rows_v1, out1_hbm.at[pl.ds(base, GB)])
            return carry

        lax.fori_loop(0, nw, step, 0)

    return pl.kernel(
        body,
        out_type=(jax.ShapeDtypeStruct((M, D), jnp.float32),
                  jax.ShapeDtypeStruct((M, D), jnp.float32)),
        mesh=_SC_MESH,
        compiler_params=pltpu.CompilerParams(use_tc_tiling_on_sc=False),
        scratch_types=[pltpu.VMEM((k, _GCH), jnp.int32),
                       pltpu.VMEM((k, _GCH), jnp.int32),
                       pltpu.VMEM((GB, D), jnp.float32),
                       pltpu.VMEM((GB, D), jnp.float32),
                       pltpu.SemaphoreType.DMA,
                       pltpu.SemaphoreType.DMA],
    )(table0, idx0, table1, idx1)


def _sc_scatter_sum(vals, idx, zeros, N):
    E, D = vals.shape
    nchunks = E // _GCH
    half = nchunks // 2
    NS = 16
    n_per_sub = N // NS

    def body(vals_hbm, idx_hbm, zeros_hbm, out_hbm, idx_v, rows_v, acc, sem):
        c = lax.axis_index("c")
        s = lax.axis_index("s")
        pltpu.sync_copy(zeros_hbm, acc.at[pl.ds(s * n_per_sub, n_per_sub)])
        plsc.subcore_barrier()

        nw = (half - s + NS - 1) // NS

        def step(k, carry):
            base = (c * half + s + k * NS) * _GCH
            pltpu.sync_copy(idx_hbm.at[pl.ds(base, _GCH)], idx_v)
            pltpu.sync_copy(vals_hbm.at[pl.ds(base, _GCH)], rows_v)
            pltpu.sync_copy(rows_v, acc.at[idx_v], add=True)
            return carry

        lax.fori_loop(0, nw, step, 0)
        plsc.subcore_barrier()
        pltpu.sync_copy(acc.at[pl.ds(s * n_per_sub, n_per_sub)],
                        out_hbm.at[c, pl.ds(s * n_per_sub, n_per_sub)])

    return pl.kernel(
        body,
        out_type=jax.ShapeDtypeStruct((2, N, D), jnp.float32),
        mesh=_SC_MESH,
        compiler_params=pltpu.CompilerParams(use_tc_tiling_on_sc=False),
        scratch_types=[pltpu.VMEM((_GCH,), jnp.int32),
                       pltpu.VMEM((_GCH, D), jnp.float32),
                       pltpu.VMEM_SHARED((N, D), jnp.float32),
                       pltpu.SemaphoreType.DMA],
    )(vals, idx, zeros)


def _sc_segsum_sorted(vals, idx, tb, zeros, E):
    T, D = vals.shape
    NS = 16
    NQ = 4
    Q = E // NQ
    QP = Q + 16
    zlen = QP // NS

    def body(vals_hbm, idx_hbm, tb_hbm, zeros_hbm, out_hbm, tb_v, idx_raw, idx_v, rows_v, acc, sem):
        c = lax.axis_index("c")
        s = lax.axis_index("s")
        pltpu.sync_copy(tb_hbm, tb_v)
        iota = lax.iota(jnp.int32, 16)
        tvec = tb_v[...]

        for j in range(2):
            t0 = jnp.where(c == 0, tvec[j], tvec[2 + j])
            t1 = jnp.where(c == 0, tvec[j + 1], tvec[3 + j])
            e0 = (c * 2 + j) * Q
            pltpu.sync_copy(zeros_hbm, acc.at[pl.ds(s * zlen, zlen)])
            plsc.subcore_barrier()

            t0a = (t0 // 8) * 8
            n = (t1 - t0a + _GCH - 1) // _GCH
            nw = (n - s + NS - 1) // NS

            def step(k, carry, t0=t0, t1=t1, e0=e0, t0a=t0a):
                nominal = t0a + (s + k * NS) * _GCH
                start = jnp.minimum(nominal, T - _GCH)
                pltpu.sync_copy(idx_hbm.at[pl.ds(start, _GCH)], idx_raw)
                pltpu.sync_copy(vals_hbm.at[pl.ds(start, _GCH)], rows_v)
                lo = jnp.maximum(t0, nominal)
                for i in range(_GCH // 16):
                    ids = idx_raw[pl.ds(i * 16, 16)]
                    tpos = start + i * 16 + iota
                    mask = (tpos >= lo) & (tpos < t1)
                    idx_v[pl.ds(i * 16, 16)] = jnp.where(mask, ids - e0, Q)
                pltpu.sync_copy(rows_v, acc.at[idx_v], add=True)
                return carry

            lax.fori_loop(0, nw, step, 0)
            plsc.subcore_barrier()
            base = s * 2496
            if_last = s == NS - 1

            @pl.when(if_last)
            def _():
                pltpu.sync_copy(acc.at[pl.ds(base, Q - 15 * 2496)],
                                out_hbm.at[pl.ds(e0 + base, Q - 15 * 2496)])

            @pl.when(jnp.logical_not(if_last))
            def _():
                pltpu.sync_copy(acc.at[pl.ds(base, 2496)],
                                out_hbm.at[pl.ds(e0 + base, 2496)])
            plsc.subcore_barrier()

    return pl.kernel(
        body,
        out_type=jax.ShapeDtypeStruct((E, D), jnp.float32),
        mesh=_SC_MESH,
        compiler_params=pltpu.CompilerParams(use_tc_tiling_on_sc=False),
        scratch_types=[pltpu.VMEM((16,), jnp.int32),
                       pltpu.VMEM((_GCH,), jnp.int32),
                       pltpu.VMEM((_GCH,), jnp.int32),
                       pltpu.VMEM((_GCH, D), jnp.float32),
                       pltpu.VMEM_SHARED((QP, D), jnp.float32),
                       pltpu.SemaphoreType.DMA],
    )(vals, idx, tb, zeros)


def _mkt_body(m_ref, rbf_ref, wmkt_ref, wrbf3_ref, wdown_ref, out_ref):
    t = _silu(jnp.dot(m_ref[...], wmkt_ref[...], preferred_element_type=jnp.float32))
    t = t * jnp.dot(rbf_ref[...], wrbf3_ref[...], preferred_element_type=jnp.float32)
    out_ref[...] = _silu(jnp.dot(t, wdown_ref[...], preferred_element_type=jnp.float32))


def _stage_mkt(m_st, rbf, W_mkt, W_rbf3, W_down):
    E, D = m_st.shape
    K = W_down.shape[1]
    return pl.pallas_call(
        _mkt_body,
        grid=(E // EB,),
        in_specs=[_row_spec(EB, D), _row_spec(EB, rbf.shape[1]),
                  _full_spec(W_mkt.shape), _full_spec(W_rbf3.shape), _full_spec(W_down.shape)],
        out_specs=_row_spec(EB, K),
        out_shape=jax.ShapeDtypeStruct((E, K), jnp.float32),
    )(m_st, rbf, W_mkt, W_rbf3, W_down)


def _bilinear_body(mg_ref, cbf_ref, s_ref, srep_ref, wb_ref, out_ref):
    C = cbf_ref.shape[1]
    cbf_exp = jnp.dot(cbf_ref[...], s_ref[...], preferred_element_type=jnp.float32)
    mg_exp = jnp.concatenate([mg_ref[...]] * C, axis=1)
    out_ref[...] = jnp.dot(cbf_exp * mg_exp, wb_ref[...], preferred_element_type=jnp.float32)


def _stage_bilinear(m_kt_g, cbf, S, Srep, Wb2):
    T, K = m_kt_g.shape
    B = Wb2.shape[1]
    return pl.pallas_call(
        _bilinear_body,
        grid=(T // TB,),
        in_specs=[_row_spec(TB, K), _row_spec(TB, cbf.shape[1]),
                  _full_spec(S.shape), _full_spec(Srep.shape), _full_spec(Wb2.shape)],
        out_specs=_row_spec(TB, B),
        out_shape=jax.ShapeDtypeStruct((T, B), jnp.float32),
    )(m_kt_g, cbf, S, Srep, Wb2)


def _xts_body(x_ref, w_ref, out_ref):
    out_ref[...] = _silu(jnp.dot(x_ref[...], w_ref[...], preferred_element_type=jnp.float32))


def _stage_xts(x, W_up_ts):
    E, B = x.shape
    D = W_up_ts.shape[1]
    return pl.pallas_call(
        _xts_body,
        grid=(E // EB,),
        in_specs=[_row_spec(EB, B), _full_spec(W_up_ts.shape)],
        out_specs=_row_spec(EB, D),
        out_shape=jax.ShapeDtypeStruct((E, D), jnp.float32),
    )(x, W_up_ts)


def _edge_chain_body(m_ref, rbf_ref, x_ref, xsw_ref,
                     wskip_ref, wupst_ref, wupts_ref, wrb0_ref, wrb1_ref, wra0_ref, wra1_ref,
                     waerbf_ref, m_out_ref, xa_pre_ref):
    m_st = m_ref[...]
    x_skip = jnp.dot(m_st, wskip_ref[...], preferred_element_type=jnp.float32)
    x_st = _silu(jnp.dot(x_ref[...], wupst_ref[...], preferred_element_type=jnp.float32))
    x_ts_sw = _silu(jnp.dot(xsw_ref[...], wupts_ref[...], preferred_element_type=jnp.float32))
    x3 = (x_st + x_ts_sw) * INV_SQRT_2
    x = (x_skip + x3) * INV_SQRT_2
    y = _silu(jnp.dot(x, wrb0_ref[...], preferred_element_type=jnp.float32))
    y = _silu(jnp.dot(y, wrb1_ref[...], preferred_element_type=jnp.float32))
    x = (x + y) * INV_SQRT_2
    m = (m_st + x) * INV_SQRT_2
    y = _silu(jnp.dot(m, wra0_ref[...], preferred_element_type=jnp.float32))
    y = _silu(jnp.dot(y, wra1_ref[...], preferred_element_type=jnp.float32))
    m = (m + y) * INV_SQRT_2
    m_out_ref[...] = m
    xa_pre_ref[...] = m * jnp.dot(rbf_ref[...], waerbf_ref[...], preferred_element_type=jnp.float32)


def _stage_edge_chain(m_st, rbf, x, x_sw, W_skip, W_up_st, W_up_ts, Wrb0, Wrb1, Wra0, Wra1, W_ae_rbf):
    E, D = m_st.shape
    return pl.pallas_call(
        _edge_chain_body,
        grid=(E // EB,),
        in_specs=[_row_spec(EB, D), _row_spec(EB, rbf.shape[1]), _row_spec(EB, x.shape[1]),
                  _row_spec(EB, x_sw.shape[1])] + [_full_spec(w.shape) for w in
                                       (W_skip, W_up_st, W_up_ts, Wrb0, Wrb1, Wra0, Wra1, W_ae_rbf)],
        out_specs=[_row_spec(EB, D), _row_spec(EB, D)],
        out_shape=[jax.ShapeDtypeStruct((E, D), jnp.float32),
                   jax.ShapeDtypeStruct((E, D), jnp.float32)],
    )(m_st, rbf, x, x_sw, W_skip, W_up_st, W_up_ts, Wrb0, Wrb1, Wra0, Wra1, W_ae_rbf)


def _node_body(xa_ref, h_ref, wd_ref, wa0_ref, wa1_ref, ws0_ref, ws1_ref,
               out_ref, hp_s_ref, hp_t_ref):
    xa = jnp.sum(xa_ref[...], axis=0)
    xa = _silu(jnp.dot(xa, wd_ref[...], preferred_element_type=jnp.float32))
    y = _silu(jnp.dot(xa, wa0_ref[...], preferred_element_type=jnp.float32))
    y = _silu(jnp.dot(y, wa1_ref[...], preferred_element_type=jnp.float32))
    xa = (xa + y) * INV_SQRT_2
    h_new = (h_ref[...] + xa) * INV_SQRT_2
    out_ref[...] = h_new
    hp_s_ref[...] = jnp.dot(h_new, ws0_ref[...], preferred_element_type=jnp.float32)
    hp_t_ref[...] = jnp.dot(h_new, ws1_ref[...], preferred_element_type=jnp.float32)


def _stage_node(xa_parts, h, W_ae_dense, Wa0, Wa1, Ws0, Ws1):
    P, N, D = xa_parts.shape
    A = h.shape[1]
    return pl.pallas_call(
        _node_body,
        grid=(N // NB,),
        in_specs=[pl.BlockSpec((P, NB, D), lambda i: (0, i, 0)), _row_spec(NB, A),
                  _full_spec(W_ae_dense.shape), _full_spec(Wa0.shape), _full_spec(Wa1.shape),
                  _full_spec(Ws0.shape), _full_spec(Ws1.shape)],
        out_specs=[_row_spec(NB, A), _row_spec(NB, D), _row_spec(NB, D)],
        out_shape=[jax.ShapeDtypeStruct((N, A), jnp.float32),
                   jax.ShapeDtypeStruct((N, D), jnp.float32),
                   jax.ShapeDtypeStruct((N, D), jnp.float32)],
    )(xa_parts, h, W_ae_dense, Wa0, Wa1, Ws0, Ws1)


def _final_body(hsp_ref, htp_ref, m_ref, ws2_ref, wm0_ref, wm1_ref, out_ref):
    m = m_ref[...]
    m2 = _silu(hsp_ref[...] + htp_ref[...]
               + jnp.dot(m, ws2_ref[...], preferred_element_type=jnp.float32))
    y = _silu(jnp.dot(m2, wm0_ref[...], preferred_element_type=jnp.float32))
    y = _silu(jnp.dot(y, wm1_ref[...], preferred_element_type=jnp.float32))
    m2 = (m2 + y) * INV_SQRT_2
    out_ref[...] = (m + m2) * INV_SQRT_2


def _stage_final(hsp, htp, m, Ws2, Wm0, Wm1):
    E, D = m.shape
    return pl.pallas_call(
        _final_body,
        grid=(E // EB,),
        in_specs=[_row_spec(EB, hsp.shape[1]), _row_spec(EB, htp.shape[1]), _row_spec(EB, D)]
                 + [_full_spec(w.shape) for w in (Ws2, Wm0, Wm1)],
        out_specs=_row_spec(EB, D),
        out_shape=jax.ShapeDtypeStruct((E, D), jnp.float32),
    )(hsp, htp, m, Ws2, Wm0, Wm1)


def kernel(h, m_st, rbf, cbf, idx_s, idx_t, idx_swap, id3_kt, id3_st, id3_ragged_idx,
           W_skip, W_mkt, W_rbf3, W_down, W_bil, W_up_st, W_up_ts,
           W_res_before, W_res_after, W_ae_rbf, W_ae_dense, W_ae_res, W_self, W_res_m):
    E = m_st.shape[0]
    N, A = h.shape
    D = m_st.shape[1]
    C, K, B = W_bil.shape

    Wb2 = W_bil.reshape(C * K, B)
    S = jnp.kron(jnp.eye(C, dtype=jnp.float32), jnp.ones((1, K), jnp.float32))
    Srep = jnp.kron(jnp.ones((1, C), jnp.float32), jnp.eye(K, dtype=jnp.float32))
    Wrb0, Wrb1 = W_res_before[0, 0], W_res_before[0, 1]
    Wra0, Wra1 = W_res_after[0, 0], W_res_after[0, 1]
    Wa0, Wa1 = W_ae_res[0, 0], W_ae_res[0, 1]
    Ws0, Ws1, Ws2 = W_self[:A], W_self[A:2 * A], W_self[2 * A:]
    Wm0, Wm1 = W_res_m[0, 0], W_res_m[0, 1]

    id3_kt = id3_kt.astype(jnp.int32)
    idx_swap = idx_swap.astype(jnp.int32)
    idx_s = idx_s.astype(jnp.int32)
    idx_t = idx_t.astype(jnp.int32)

    id3_st = id3_st.astype(jnp.int32)
    Q = E // 4
    tb = jnp.searchsorted(id3_st, jnp.arange(5, dtype=jnp.int32) * Q).astype(jnp.int32)
    tb = jnp.concatenate([tb, jnp.zeros((11,), jnp.int32)])
    zeros_n = jnp.zeros((N // 16, D), jnp.float32)
    zeros_q = jnp.zeros(((Q + 16) // 16, B), jnp.float32)

    id3_kt2 = id3_kt.reshape(-1, _GCH)
    idx_swap2 = idx_swap.reshape(-1, _GCH)
    idx_s2 = idx_s.reshape(-1, _GCH)
    idx_t2 = idx_t.reshape(-1, _GCH)

    m_kt = _stage_mkt(m_st, rbf, W_mkt, W_rbf3, W_down)
    m_kt_g = _sc_gather(m_kt, id3_kt2, k=10)
    v = _stage_bilinear(m_kt_g, cbf, S, Srep, Wb2)
    x = _sc_segsum_sorted(v, id3_st, tb, zeros_q, E)
    x_sw = _sc_gather(x, idx_swap2, k=10)
    m_mid, xa_pre = _stage_edge_chain(m_st, rbf, x, x_sw,
                                      W_skip, W_up_st, W_up_ts, Wrb0, Wrb1, Wra0, Wra1, W_ae_rbf)
    xa_parts = _sc_scatter_sum(xa_pre, idx_t, zeros_n, N)
    h_new, hp_s, hp_t = _stage_node(xa_parts, h, W_ae_dense, Wa0, Wa1, Ws0, Ws1)
    hsp, htp = _sc_gather2(hp_s, idx_s2, hp_t, idx_t2, k=2)
    m_new = _stage_final(hsp, htp, m_mid, Ws2, Wm0, Wm1)
    return h_new, m_new


# speedup vs baseline: 4.3388x; 1.0306x over previous
import functools

import jax
import jax.numpy as jnp
from jax import lax
from jax.experimental import pallas as pl
from jax.experimental.pallas import tpu as pltpu
from jax.experimental.pallas import tpu_sc as plsc

INV_SQRT_2 = 0.7071067811865475

_SC_MESH = plsc.VectorSubcoreMesh(core_axis_name="c", subcore_axis_name="s")
_NW = 32
_GCH = 128


def _sc_gather(table, idx, k):
    M = idx.shape[0] * _GCH
    D = table.shape[1]
    GB = _GCH * k
    nchunks = M // GB
    assert M % GB == 0

    def body(table_hbm, idx_hbm, out_hbm, idx_v, rows_v, sem):
        w = lax.axis_index("s") * 2 + lax.axis_index("c")
        nw = (nchunks - w + _NW - 1) // _NW

        def step(i, carry):
            c = w + i * _NW
            base = c * GB
            pltpu.sync_copy(idx_hbm.at[pl.ds(c * k, k)], idx_v)
            cps = [pltpu.async_copy(table_hbm.at[idx_v.at[j]],
                                    rows_v.at[pl.ds(j * _GCH, _GCH)], sem)
                   for j in range(k)]
            for cp in cps:
                cp.wait()
            pltpu.sync_copy(rows_v, out_hbm.at[pl.ds(base, GB)])
            return carry

        lax.fori_loop(0, nw, step, 0)

    return pl.kernel(
        body,
        out_type=jax.ShapeDtypeStruct((M, D), jnp.float32),
        mesh=_SC_MESH,
        compiler_params=pltpu.CompilerParams(use_tc_tiling_on_sc=False),
        scratch_types=[pltpu.VMEM((k, _GCH), jnp.int32),
                       pltpu.VMEM((GB, D), jnp.float32),
                       pltpu.SemaphoreType.DMA],
    )(table, idx)

EB = 1280
TB = 1280
NB = 2000


def _silu(x):
    return x * (1.0 / (1.0 + jnp.exp(-x)))


def _row_spec(rb, cols):
    return pl.BlockSpec((rb, cols), lambda i: (i, 0))


def _full_spec(shape):
    nd = len(shape)
    return pl.BlockSpec(shape, lambda i: (0,) * nd)


def _sc_gather2(table0, idx0, table1, idx1, k=2):
    M = idx0.shape[0] * _GCH
    D = table0.shape[1]
    GB = _GCH * k
    nchunks = M // GB
    assert M % GB == 0

    def body(t0_hbm, i0_hbm, t1_hbm, i1_hbm, out0_hbm, out1_hbm,
             idx_v0, idx_v1, rows_v0, rows_v1, sem0, sem1):
        w = lax.axis_index("s") * 2 + lax.axis_index("c")
        nw = (nchunks - w + _NW - 1) // _NW

        def step(i, carry):
            c = w + i * _NW
            base = c * GB
            pltpu.sync_copy(i0_hbm.at[pl.ds(c * k, k)], idx_v0)
            pltpu.sync_copy(i1_hbm.at[pl.ds(c * k, k)], idx_v1)
            cp0 = [pltpu.async_copy(t0_hbm.at[idx_v0.at[j]],
                                    rows_v0.at[pl.ds(j * _GCH, _GCH)], sem0)
                   for j in range(k)]
            cp1 = [pltpu.async_copy(t1_hbm.at[idx_v1.at[j]],
                                    rows_v1.at[pl.ds(j * _GCH, _GCH)], sem1)
                   for j in range(k)]
            for cp in cp0:
                cp.wait()
            pltpu.sync_copy(rows_v0, out0_hbm.at[pl.ds(base, GB)])
            for cp in cp1:
                cp.wait()
            pltpu.sync_copy(rows_v1, out1_hbm.at[pl.ds(base, GB)])
            return carry

        lax.fori_loop(0, nw, step, 0)

    return pl.kernel(
        body,
        out_type=(jax.ShapeDtypeStruct((M, D), jnp.float32),
                  jax.ShapeDtypeStruct((M, D), jnp.float32)),
        mesh=_SC_MESH,
        compiler_params=pltpu.CompilerParams(use_tc_tiling_on_sc=False),
        scratch_types=[pltpu.VMEM((k, _GCH), jnp.int32),
                       pltpu.VMEM((k, _GCH), jnp.int32),
                       pltpu.VMEM((GB, D), jnp.float32),
                       pltpu.VMEM((GB, D), jnp.float32),
                       pltpu.SemaphoreType.DMA,
                       pltpu.SemaphoreType.DMA],
    )(table0, idx0, table1, idx1)


def _sc_scatter_sum(vals, idx2, zeros, N, k=1):
    E, D = vals.shape
    GB = _GCH * k
    nchunks = E // GB
    half = nchunks // 2
    NS = 16
    n_per_sub = N // NS

    def body(vals_hbm, idx_hbm, zeros_hbm, out_hbm, idx_v, rows_v, acc, sem):
        c = lax.axis_index("c")
        s = lax.axis_index("s")
        pltpu.sync_copy(zeros_hbm, acc.at[pl.ds(s * n_per_sub, n_per_sub)])
        plsc.subcore_barrier()

        nw = (half - s + NS - 1) // NS

        def step(kk, carry):
            cidx = c * half + s + kk * NS
            pltpu.sync_copy(idx_hbm.at[pl.ds(cidx * k, k)], idx_v)
            pltpu.sync_copy(vals_hbm.at[pl.ds(cidx * GB, GB)], rows_v)
            for j in range(k):
                pltpu.sync_copy(rows_v.at[pl.ds(j * _GCH, _GCH)],
                                acc.at[idx_v.at[j]], add=True)
            return carry

        lax.fori_loop(0, nw, step, 0)
        plsc.subcore_barrier()
        pltpu.sync_copy(acc.at[pl.ds(s * n_per_sub, n_per_sub)],
                        out_hbm.at[c, pl.ds(s * n_per_sub, n_per_sub)])

    return pl.kernel(
        body,
        out_type=jax.ShapeDtypeStruct((2, N, D), jnp.float32),
        mesh=_SC_MESH,
        compiler_params=pltpu.CompilerParams(use_tc_tiling_on_sc=False),
        scratch_types=[pltpu.VMEM((k, _GCH), jnp.int32),
                       pltpu.VMEM((GB, D), jnp.float32),
                       pltpu.VMEM_SHARED((N, D), jnp.float32),
                       pltpu.SemaphoreType.DMA],
    )(vals, idx2, zeros)


def _sc_segsum_sorted(vals, idx, tb, zeros, E):
    T, D = vals.shape
    NS = 16
    NQ = 8
    KC = 4
    Q = E // NQ
    QP = Q + 16
    zlen = QP // NS
    WCH = 1248

    def body(vals_hbm, idx_hbm, tb_hbm, zeros_hbm, out_hbm, tb_v, idx_raw, idx_v, rows_v, acc, sem):
        c = lax.axis_index("c")
        s = lax.axis_index("s")
        pltpu.sync_copy(tb_hbm, tb_v)
        iota = lax.iota(jnp.int32, 16)
        tvec = tb_v[...]

        for j in range(NQ // 2):
            t0 = jnp.where(c == 0, tvec[j], tvec[NQ // 2 + j])
            t1 = jnp.where(c == 0, tvec[j + 1], tvec[NQ // 2 + j + 1])
            e0 = (c * (NQ // 2) + j) * Q
            pltpu.sync_copy(zeros_hbm, acc.at[pl.ds(s * zlen, zlen)])
            plsc.subcore_barrier()

            t0a = (t0 // 8) * 8
            GB = _GCH * KC
            n = (t1 - t0a + GB - 1) // GB
            nw = (n - s + NS - 1) // NS

            def step(k, carry, t0=t0, t1=t1, e0=e0, t0a=t0a):
                nominal = t0a + (s + k * NS) * GB
                start = jnp.minimum(nominal, T - GB)
                pltpu.sync_copy(idx_hbm.at[pl.ds(start, GB)], idx_raw)
                pltpu.sync_copy(vals_hbm.at[pl.ds(start, GB)], rows_v)
                lo = jnp.maximum(t0, nominal)
                for j in range(KC):
                    for i in range(_GCH // 16):
                        ids = idx_raw[pl.ds(j * _GCH + i * 16, 16)]
                        tpos = start + j * _GCH + i * 16 + iota
                        mask = (tpos >= lo) & (tpos < t1)
                        idx_v[j, pl.ds(i * 16, 16)] = jnp.where(mask, ids - e0, Q)
                for j in range(KC):
                    pltpu.sync_copy(rows_v.at[pl.ds(j * _GCH, _GCH)],
                                    acc.at[idx_v.at[j]], add=True)
                return carry

            lax.fori_loop(0, nw, step, 0)
            plsc.subcore_barrier()
            base = s * WCH
            if_last = s == NS - 1

            @pl.when(if_last)
            def _():
                pltpu.sync_copy(acc.at[pl.ds(base, Q - 15 * WCH)],
                                out_hbm.at[pl.ds(e0 + base, Q - 15 * WCH)])

            @pl.when(jnp.logical_not(if_last))
            def _():
                pltpu.sync_copy(acc.at[pl.ds(base, WCH)],
                                out_hbm.at[pl.ds(e0 + base, WCH)])
            plsc.subcore_barrier()

    return pl.kernel(
        body,
        out_type=jax.ShapeDtypeStruct((E, D), jnp.float32),
        mesh=_SC_MESH,
        compiler_params=pltpu.CompilerParams(use_tc_tiling_on_sc=False),
        scratch_types=[pltpu.VMEM((16,), jnp.int32),
                       pltpu.VMEM((_GCH * KC,), jnp.int32),
                       pltpu.VMEM((KC, _GCH), jnp.int32),
                       pltpu.VMEM((_GCH * KC, D), jnp.float32),
                       pltpu.VMEM_SHARED((QP, D), jnp.float32),
                       pltpu.SemaphoreType.DMA],
    )(vals, idx, tb, zeros)


def _mkt_body(m_ref, rbf_ref, wmkt_ref, wrbf3_ref, wdown_ref, out_ref):
    t = _silu(jnp.dot(m_ref[...], wmkt_ref[...], preferred_element_type=jnp.float32))
    t = t * jnp.dot(rbf_ref[...], wrbf3_ref[...], preferred_element_type=jnp.float32)
    out_ref[...] = _silu(jnp.dot(t, wdown_ref[...], preferred_element_type=jnp.float32))


def _stage_mkt(m_st, rbf, W_mkt, W_rbf3, W_down):
    E, D = m_st.shape
    K = W_down.shape[1]
    return pl.pallas_call(
        _mkt_body,
        grid=(E // EB,),
        in_specs=[_row_spec(EB, D), _row_spec(EB, rbf.shape[1]),
                  _full_spec(W_mkt.shape), _full_spec(W_rbf3.shape), _full_spec(W_down.shape)],
        out_specs=_row_spec(EB, K),
        out_shape=jax.ShapeDtypeStruct((E, K), jnp.float32),
    )(m_st, rbf, W_mkt, W_rbf3, W_down)


def _bilinear_body(mg_ref, cbf_ref, s_ref, srep_ref, wb_ref, out_ref):
    C = cbf_ref.shape[1]
    cbf_exp = jnp.dot(cbf_ref[...], s_ref[...], preferred_element_type=jnp.float32)
    mg_exp = jnp.concatenate([mg_ref[...]] * C, axis=1)
    out_ref[...] = jnp.dot(cbf_exp * mg_exp, wb_ref[...], preferred_element_type=jnp.float32)


def _stage_bilinear(m_kt_g, cbf, S, Srep, Wb2):
    T, K = m_kt_g.shape
    B = Wb2.shape[1]
    return pl.pallas_call(
        _bilinear_body,
        grid=(T // TB,),
        in_specs=[_row_spec(TB, K), _row_spec(TB, cbf.shape[1]),
                  _full_spec(S.shape), _full_spec(Srep.shape), _full_spec(Wb2.shape)],
        out_specs=_row_spec(TB, B),
        out_shape=jax.ShapeDtypeStruct((T, B), jnp.float32),
    )(m_kt_g, cbf, S, Srep, Wb2)


def _xts_body(x_ref, w_ref, out_ref):
    out_ref[...] = _silu(jnp.dot(x_ref[...], w_ref[...], preferred_element_type=jnp.float32))


def _stage_xts(x, W_up_ts):
    E, B = x.shape
    D = W_up_ts.shape[1]
    return pl.pallas_call(
        _xts_body,
        grid=(E // EB,),
        in_specs=[_row_spec(EB, B), _full_spec(W_up_ts.shape)],
        out_specs=_row_spec(EB, D),
        out_shape=jax.ShapeDtypeStruct((E, D), jnp.float32),
    )(x, W_up_ts)


def _edge_chain_body(m_ref, rbf_ref, x_ref, xsw_ref,
                     wskip_ref, wupst_ref, wupts_ref, wrb0_ref, wrb1_ref, wra0_ref, wra1_ref,
                     waerbf_ref, m_out_ref, xa_pre_ref):
    m_st = m_ref[...]
    x_skip = jnp.dot(m_st, wskip_ref[...], preferred_element_type=jnp.float32)
    x_st = _silu(jnp.dot(x_ref[...], wupst_ref[...], preferred_element_type=jnp.float32))
    x_ts_sw = _silu(jnp.dot(xsw_ref[...], wupts_ref[...], preferred_element_type=jnp.float32))
    x3 = (x_st + x_ts_sw) * INV_SQRT_2
    x = (x_skip + x3) * INV_SQRT_2
    y = _silu(jnp.dot(x, wrb0_ref[...], preferred_element_type=jnp.float32))
    y = _silu(jnp.dot(y, wrb1_ref[...], preferred_element_type=jnp.float32))
    x = (x + y) * INV_SQRT_2
    m = (m_st + x) * INV_SQRT_2
    y = _silu(jnp.dot(m, wra0_ref[...], preferred_element_type=jnp.float32))
    y = _silu(jnp.dot(y, wra1_ref[...], preferred_element_type=jnp.float32))
    m = (m + y) * INV_SQRT_2
    m_out_ref[...] = m
    xa_pre_ref[...] = m * jnp.dot(rbf_ref[...], waerbf_ref[...], preferred_element_type=jnp.float32)


def _stage_edge_chain(m_st, rbf, x, x_sw, W_skip, W_up_st, W_up_ts, Wrb0, Wrb1, Wra0, Wra1, W_ae_rbf):
    E, D = m_st.shape
    return pl.pallas_call(
        _edge_chain_body,
        grid=(E // EB,),
        in_specs=[_row_spec(EB, D), _row_spec(EB, rbf.shape[1]), _row_spec(EB, x.shape[1]),
                  _row_spec(EB, x_sw.shape[1])] + [_full_spec(w.shape) for w in
                                       (W_skip, W_up_st, W_up_ts, Wrb0, Wrb1, Wra0, Wra1, W_ae_rbf)],
        out_specs=[_row_spec(EB, D), _row_spec(EB, D)],
        out_shape=[jax.ShapeDtypeStruct((E, D), jnp.float32),
                   jax.ShapeDtypeStruct((E, D), jnp.float32)],
    )(m_st, rbf, x, x_sw, W_skip, W_up_st, W_up_ts, Wrb0, Wrb1, Wra0, Wra1, W_ae_rbf)


def _node_body(xa_ref, h_ref, wd_ref, wa0_ref, wa1_ref, ws0_ref, ws1_ref,
               out_ref, hp_s_ref, hp_t_ref):
    xa = jnp.sum(xa_ref[...], axis=0)
    xa = _silu(jnp.dot(xa, wd_ref[...], preferred_element_type=jnp.float32))
    y = _silu(jnp.dot(xa, wa0_ref[...], preferred_element_type=jnp.float32))
    y = _silu(jnp.dot(y, wa1_ref[...], preferred_element_type=jnp.float32))
    xa = (xa + y) * INV_SQRT_2
    h_new = (h_ref[...] + xa) * INV_SQRT_2
    out_ref[...] = h_new
    hp_s_ref[...] = jnp.dot(h_new, ws0_ref[...], preferred_element_type=jnp.float32)
    hp_t_ref[...] = jnp.dot(h_new, ws1_ref[...], preferred_element_type=jnp.float32)


def _stage_node(xa_parts, h, W_ae_dense, Wa0, Wa1, Ws0, Ws1):
    P, N, D = xa_parts.shape
    A = h.shape[1]
    return pl.pallas_call(
        _node_body,
        grid=(N // NB,),
        in_specs=[pl.BlockSpec((P, NB, D), lambda i: (0, i, 0)), _row_spec(NB, A),
                  _full_spec(W_ae_dense.shape), _full_spec(Wa0.shape), _full_spec(Wa1.shape),
                  _full_spec(Ws0.shape), _full_spec(Ws1.shape)],
        out_specs=[_row_spec(NB, A), _row_spec(NB, D), _row_spec(NB, D)],
        out_shape=[jax.ShapeDtypeStruct((N, A), jnp.float32),
                   jax.ShapeDtypeStruct((N, D), jnp.float32),
                   jax.ShapeDtypeStruct((N, D), jnp.float32)],
    )(xa_parts, h, W_ae_dense, Wa0, Wa1, Ws0, Ws1)


def _final_body(hsp_ref, htp_ref, m_ref, ws2_ref, wm0_ref, wm1_ref, out_ref):
    m = m_ref[...]
    m2 = _silu(hsp_ref[...] + htp_ref[...]
               + jnp.dot(m, ws2_ref[...], preferred_element_type=jnp.float32))
    y = _silu(jnp.dot(m2, wm0_ref[...], preferred_element_type=jnp.float32))
    y = _silu(jnp.dot(y, wm1_ref[...], preferred_element_type=jnp.float32))
    m2 = (m2 + y) * INV_SQRT_2
    out_ref[...] = (m + m2) * INV_SQRT_2


def _stage_final(hsp, htp, m, Ws2, Wm0, Wm1):
    E, D = m.shape
    return pl.pallas_call(
        _final_body,
        grid=(E // EB,),
        in_specs=[_row_spec(EB, hsp.shape[1]), _row_spec(EB, htp.shape[1]), _row_spec(EB, D)]
                 + [_full_spec(w.shape) for w in (Ws2, Wm0, Wm1)],
        out_specs=_row_spec(EB, D),
        out_shape=jax.ShapeDtypeStruct((E, D), jnp.float32),
    )(hsp, htp, m, Ws2, Wm0, Wm1)


def kernel(h, m_st, rbf, cbf, idx_s, idx_t, idx_swap, id3_kt, id3_st, id3_ragged_idx,
           W_skip, W_mkt, W_rbf3, W_down, W_bil, W_up_st, W_up_ts,
           W_res_before, W_res_after, W_ae_rbf, W_ae_dense, W_ae_res, W_self, W_res_m):
    E = m_st.shape[0]
    N, A = h.shape
    D = m_st.shape[1]
    C, K, B = W_bil.shape

    Wb2 = W_bil.reshape(C * K, B)
    S = jnp.kron(jnp.eye(C, dtype=jnp.float32), jnp.ones((1, K), jnp.float32))
    Srep = jnp.kron(jnp.ones((1, C), jnp.float32), jnp.eye(K, dtype=jnp.float32))
    Wrb0, Wrb1 = W_res_before[0, 0], W_res_before[0, 1]
    Wra0, Wra1 = W_res_after[0, 0], W_res_after[0, 1]
    Wa0, Wa1 = W_ae_res[0, 0], W_ae_res[0, 1]
    Ws0, Ws1, Ws2 = W_self[:A], W_self[A:2 * A], W_self[2 * A:]
    Wm0, Wm1 = W_res_m[0, 0], W_res_m[0, 1]

    id3_kt = id3_kt.astype(jnp.int32)
    idx_swap = idx_swap.astype(jnp.int32)
    idx_s = idx_s.astype(jnp.int32)
    idx_t = idx_t.astype(jnp.int32)

    id3_st = id3_st.astype(jnp.int32)
    Q = E // 8
    tb = jnp.searchsorted(id3_st, jnp.arange(9, dtype=jnp.int32) * Q).astype(jnp.int32)
    tb = jnp.concatenate([tb, jnp.zeros((7,), jnp.int32)])
    zeros_n = jnp.zeros((N // 16, D), jnp.float32)
    zeros_q = jnp.zeros(((Q + 16) // 16, B), jnp.float32)

    id3_kt2 = id3_kt.reshape(-1, _GCH)
    idx_swap2 = idx_swap.reshape(-1, _GCH)
    idx_s2 = idx_s.reshape(-1, _GCH)
    idx_t2 = idx_t.reshape(-1, _GCH)

    m_kt = _stage_mkt(m_st, rbf, W_mkt, W_rbf3, W_down)
    m_kt_g = _sc_gather(m_kt, id3_kt2, k=10)
    v = _stage_bilinear(m_kt_g, cbf, S, Srep, Wb2)
    x = _sc_segsum_sorted(v, id3_st, tb, zeros_q, E)
    x_sw = _sc_gather(x, idx_swap2, k=10)
    m_mid, xa_pre = _stage_edge_chain(m_st, rbf, x, x_sw,
                                      W_skip, W_up_st, W_up_ts, Wrb0, Wrb1, Wra0, Wra1, W_ae_rbf)
    xa_parts = _sc_scatter_sum(xa_pre, idx_t2, zeros_n, N)
    h_new, hp_s, hp_t = _stage_node(xa_parts, h, W_ae_dense, Wa0, Wa1, Ws0, Ws1)
    hsp, htp = _sc_gather2(hp_s, idx_s2, hp_t, idx_t2, k=2)
    m_new = _stage_final(hsp, htp, m_mid, Ws2, Wm0, Wm1)
    return h_new, m_new
